# baseline (speedup 1.0000x reference)
import jax
import jax.numpy as jnp
import numpy as np
from jax.experimental import pallas as pl

N = 50000
B = 64
E = 800000
EMB = 64
AUX = 4
RH = 32
LAY = 2
BP = 3


def _normalize(x, axis=-1, eps=1e-12):
    n = jnp.linalg.norm(x, axis=axis, keepdims=True)
    return x / jnp.maximum(n, eps)


def _combine_kernel(w_ref, q0_ref, q1_ref, out_ref):
    w = jax.nn.softmax(w_ref[...], axis=1)
    out_ref[...] = w[:, 0:1] * q0_ref[...] + w[:, 1:2] * q1_ref[...]


def kernel(edge_index, graph_ids, action_nodes, aux_input, w_n2l, p_node_conv, p_node_conv2, p_node_conv3, h1_weight, h2_weight, cross_product, w_layer1, w_layer2, W_att):
    act = jax.nn.relu
    embeds = []
    for l in range(LAY):
        src = edge_index[l, 0]
        dst = edge_index[l, 1]
        deg = jax.ops.segment_sum(jnp.ones((E,), dtype=jnp.float32), src, num_segments=N)
        deg_max = jax.ops.segment_max(deg, graph_ids, num_segments=B)
        deg_new = deg / deg_max[graph_ids]
        node_input = jnp.stack([deg_new, deg_new], axis=1)
        y_node_input = jnp.ones((B, 2), dtype=jnp.float32)
        cur = _normalize(act(node_input @ w_n2l), axis=1)
        y_cur = _normalize(act(y_node_input @ w_n2l), axis=1)
        for _ in range(BP):
            n2npool = jax.ops.segment_sum(cur[dst], src, num_segments=N)
            node_linear = n2npool @ p_node_conv
            y_n2npool = jax.ops.segment_sum(cur, graph_ids, num_segments=B)
            y_node_linear = y_n2npool @ p_node_conv
            merged = jnp.concatenate([node_linear, cur @ p_node_conv2], axis=1)
            new_cur = _normalize(act(merged @ p_node_conv3), axis=1)
            y_merged = jnp.concatenate([y_node_linear, y_cur @ p_node_conv2], axis=1)
            y_cur = _normalize(act(y_merged @ p_node_conv3), axis=1)
            cur = new_cur
        embeds.append(jnp.concatenate([cur, y_cur], axis=0))
    scale = 1.0 / np.sqrt(EMB)
    msgs = []
    for l in range(LAY):
        qv = embeds[l]
        sc = jnp.stack([jnp.sum(qv * (embeds[j] @ W_att), axis=1) * scale for j in range(LAY)], axis=1)
        al = jax.nn.softmax(sc, axis=1)
        msgs.append(al[:, 0:1] * embeds[0] + al[:, 1:2] * embeds[1])
    message_layer = jnp.stack(msgs, axis=0)
    cur_msg = _normalize(message_layer[:, :N, :], axis=2)
    y_msg = _normalize(message_layer[:, N:, :], axis=2)
    q_list = []
    w_list = []
    for l in range(LAY):
        y_pot = y_msg[l]
        action_embed = cur_msg[l][action_nodes]
        temp = jnp.einsum('bi,bj->bij', action_embed, y_pot)
        embed_s_a = jnp.einsum('bij,jk->bik', temp, cross_product)[:, :, 0]
        hidden = act(embed_s_a @ h1_weight)
        last_output = jnp.concatenate([hidden, aux_input[:, l, :]], axis=1)
        q_list.append(last_output @ h2_weight)
        w_list.append(act(y_pot @ w_layer1) @ w_layer2)
    wcat = jnp.concatenate(w_list, axis=1)
    q = pl.pallas_call(
        _combine_kernel,
        out_shape=jax.ShapeDtypeStruct((B, 1), jnp.float32),
    )(wcat, q_list[0], q_list[1])
    return (q, cur_msg)


# speedup vs baseline: 5.6994x; 5.6994x over previous
import functools

import jax
import jax.numpy as jnp
import numpy as np
from jax import lax
from jax.experimental import pallas as pl
from jax.experimental.pallas import tpu as pltpu
from jax.experimental.pallas import tpu_sc as plsc

N = 50000
B = 64
E = 800000
EMB = 64
AUX = 4
RH = 32
LAY = 2
BP = 3

H = 32
NC = 2
NS = 16
CH = 1000
CHS = 400
RPT = N // NS
EPT = E // NS
EPT2 = E // (NC * NS)
BN = 5000
NB = N // BN

_EPS = 1e-12

_sc_mesh = plsc.VectorSubcoreMesh(core_axis_name="c", subcore_axis_name="s")
_sc_params = pltpu.CompilerParams(use_tc_tiling_on_sc=False,
                                  internal_scratch_in_bytes=0)


def _zero_rows(buf, nrows, ncols):
    zv = jnp.zeros((16,), jnp.float32)

    def body(i, _):
        for j in range(ncols // 16):
            buf[i, pl.ds(j * 16, 16)] = zv
        return 0

    lax.fori_loop(0, nrows, body, 0)


def _fill_ones(buf, nrows, ncols):
    ov = jnp.ones((16,), jnp.float32)

    def body(i, _):
        for j in range(ncols // 16):
            buf[i, pl.ds(j * 16, 16)] = ov
        return 0

    lax.fori_loop(0, nrows, body, 0)


def _hist_body(src0, src1, out, ones_v, idx_v, acc):
    cid = lax.axis_index("c")
    sid = lax.axis_index("s")
    srcs = [src0, src1]
    r0 = sid * RPT
    wid = cid * NS + sid
    for l in range(LAY):
        _zero_rows(ones_v, CH, 16)
        for k in range(4):
            sz = CH if k < 3 else RPT - 3 * CH
            pltpu.sync_copy(ones_v.at[pl.ds(0, sz)], acc.at[pl.ds(r0 + k * CH, sz)])
        _fill_ones(ones_v, CH, 16)
        plsc.subcore_barrier()

        def chunk(ci, _):
            base = wid * EPT2 + ci * CH
            pltpu.sync_copy(srcs[l].at[pl.ds(base, CH)], idx_v)
            pltpu.sync_copy(ones_v, acc.at[idx_v], add=True)
            return 0

        lax.fori_loop(0, EPT2 // CH, chunk, 0)
        plsc.subcore_barrier()
        pltpu.sync_copy(acc.at[pl.ds(r0, RPT)], out.at[l, cid, pl.ds(r0, RPT)])
        plsc.subcore_barrier()


_hist_call = pl.kernel(
    _hist_body,
    out_type=jax.ShapeDtypeStruct((LAY, NC, N, 16), jnp.float32),
    mesh=_sc_mesh,
    compiler_params=_sc_params,
    scratch_types=[
        pltpu.VMEM((CH, 16), jnp.float32),
        pltpu.VMEM((CH,), jnp.int32),
        pltpu.VMEM_SHARED((N, 16), jnp.float32),
    ],
)


def _spmm_body(dst, src, tlo, thi, outlo, outhi, idx_d, idx_s, rows, acc):
    cid = lax.axis_index("c")
    sid = lax.axis_index("s")
    _zero_rows(rows, CHS, H)
    r0 = sid * RPT
    nz = RPT // CHS
    for k in range(nz + 1):
        sz = CHS if k < nz else RPT - nz * CHS
        pltpu.sync_copy(rows.at[pl.ds(0, sz)], acc.at[pl.ds(r0 + k * CHS, sz)])
    plsc.subcore_barrier()

    def chunk(ci, _):
        base = sid * EPT + ci * CHS
        pltpu.sync_copy(dst.at[pl.ds(base, CHS)], idx_d)
        pltpu.sync_copy(src.at[pl.ds(base, CHS)], idx_s)

        @pl.when(cid == 0)
        def _():
            pltpu.sync_copy(tlo.at[idx_d], rows)

        @pl.when(cid == 1)
        def _():
            pltpu.sync_copy(thi.at[idx_d], rows)

        pltpu.sync_copy(rows, acc.at[idx_s], add=True)
        return 0

    lax.fori_loop(0, EPT // CHS, chunk, 0)
    plsc.subcore_barrier()

    @pl.when(cid == 0)
    def _():
        pltpu.sync_copy(acc.at[pl.ds(r0, RPT)], outlo.at[pl.ds(r0, RPT)])

    @pl.when(cid == 1)
    def _():
        pltpu.sync_copy(acc.at[pl.ds(r0, RPT)], outhi.at[pl.ds(r0, RPT)])


_spmm_call = pl.kernel(
    _spmm_body,
    out_type=(
        jax.ShapeDtypeStruct((N, H), jnp.float32),
        jax.ShapeDtypeStruct((N, H), jnp.float32),
    ),
    mesh=_sc_mesh,
    compiler_params=_sc_params,
    scratch_types=[
        pltpu.VMEM((CHS,), jnp.int32),
        pltpu.VMEM((CHS,), jnp.int32),
        pltpu.VMEM((CHS, H), jnp.float32),
        pltpu.VMEM_SHARED((N, H), jnp.float32),
    ],
)


def _norm_rows(z):
    n = jnp.sqrt(jnp.sum(z * z, axis=1, keepdims=True))
    return z / jnp.maximum(n, _EPS)


def _prep_kernel(hist_ref, oh_ref, wpad_ref, c0lo_ref, c0hi_ref, ypool0_ref, ycur0_ref):
    i = pl.program_id(0)
    w = wpad_ref[...]
    u = _norm_rows(jax.nn.relu(w[0:1, :] + w[1:2, :]))
    oh = oh_ref[...]
    ones_row = jnp.ones((1, EMB), jnp.float32)
    for l in range(LAY):
        d = hist_ref[l, 0] + hist_ref[l, 1]
        dsum = jnp.sum(d, axis=1, keepdims=True)
        mask = (dsum > 0).astype(jnp.float32)
        cur0 = mask * u
        c0lo_ref[l] = cur0[:, :H]
        c0hi_ref[l] = cur0[:, H:]
        mask64 = mask * ones_row
        cnt = lax.dot_general(oh, mask64, (((0,), (0,)), ((), ())),
                              preferred_element_type=jnp.float32)

        @pl.when(i == 0)
        def _():
            ypool0_ref[l] = cnt * u

        @pl.when(i != 0)
        def _():
            ypool0_ref[l] += cnt * u

    @pl.when(i == 0)
    def _():
        ycur0_ref[...] = jnp.ones((B, 1), jnp.float32) * u


def _prep_call(hist, oh, wpad):
    return pl.pallas_call(
        _prep_kernel,
        grid=(NB,),
        in_specs=[
            pl.BlockSpec((LAY, NC, BN, 16), lambda i: (0, 0, i, 0)),
            pl.BlockSpec((BN, EMB), lambda i: (i, 0)),
            pl.BlockSpec((8, EMB), lambda i: (0, 0)),
        ],
        out_specs=[
            pl.BlockSpec((LAY, BN, H), lambda i: (0, i, 0)),
            pl.BlockSpec((LAY, BN, H), lambda i: (0, i, 0)),
            pl.BlockSpec((LAY, B, EMB), lambda i: (0, 0, 0)),
            pl.BlockSpec((B, EMB), lambda i: (0, 0)),
        ],
        out_shape=[
            jax.ShapeDtypeStruct((LAY, N, H), jnp.float32),
            jax.ShapeDtypeStruct((LAY, N, H), jnp.float32),
            jax.ShapeDtypeStruct((LAY, B, EMB), jnp.float32),
            jax.ShapeDtypeStruct((B, EMB), jnp.float32),
        ],
    )(hist, oh, wpad)


def _dense_kernel(nplo_ref, nphi_ref, clo_ref, chi_ref, oh_ref, w1_ref, w2_ref,
                  ypool_ref, ycur_ref,
                  nlo_ref, nhi_ref, ypooln_ref, ycurn_ref):
    i = pl.program_id(0)
    w1 = w1_ref[...]
    w2 = w2_ref[...]
    np64 = jnp.concatenate([nplo_ref[...], nphi_ref[...]], axis=1)
    cur64 = jnp.concatenate([clo_ref[...], chi_ref[...]], axis=1)
    z = jax.nn.relu(
        jnp.dot(np64, w1, preferred_element_type=jnp.float32)
        + jnp.dot(cur64, w2, preferred_element_type=jnp.float32))
    new = _norm_rows(z)
    nlo_ref[...] = new[:, :H]
    nhi_ref[...] = new[:, H:]
    ypn = lax.dot_general(oh_ref[...], new, (((0,), (0,)), ((), ())),
                          preferred_element_type=jnp.float32)

    @pl.when(i == 0)
    def _():
        ypooln_ref[...] = ypn
        yz = jax.nn.relu(
            jnp.dot(ypool_ref[...], w1, preferred_element_type=jnp.float32)
            + jnp.dot(ycur_ref[...], w2, preferred_element_type=jnp.float32))
        ycurn_ref[...] = _norm_rows(yz)

    @pl.when(i != 0)
    def _():
        ypooln_ref[...] += ypn


def _dense_call(nplo, nphi, clo, chi, oh, w1, w2, ypool, ycur):
    return pl.pallas_call(
        _dense_kernel,
        grid=(NB,),
        in_specs=[
            pl.BlockSpec((BN, H), lambda i: (i, 0)),
            pl.BlockSpec((BN, H), lambda i: (i, 0)),
            pl.BlockSpec((BN, H), lambda i: (i, 0)),
            pl.BlockSpec((BN, H), lambda i: (i, 0)),
            pl.BlockSpec((BN, EMB), lambda i: (i, 0)),
            pl.BlockSpec((EMB, EMB), lambda i: (0, 0)),
            pl.BlockSpec((EMB, EMB), lambda i: (0, 0)),
            pl.BlockSpec((B, EMB), lambda i: (0, 0)),
            pl.BlockSpec((B, EMB), lambda i: (0, 0)),
        ],
        out_specs=[
            pl.BlockSpec((BN, H), lambda i: (i, 0)),
            pl.BlockSpec((BN, H), lambda i: (i, 0)),
            pl.BlockSpec((B, EMB), lambda i: (0, 0)),
            pl.BlockSpec((B, EMB), lambda i: (0, 0)),
        ],
        out_shape=[
            jax.ShapeDtypeStruct((N, H), jnp.float32),
            jax.ShapeDtypeStruct((N, H), jnp.float32),
            jax.ShapeDtypeStruct((B, EMB), jnp.float32),
            jax.ShapeDtypeStruct((B, EMB), jnp.float32),
        ],
    )(nplo, nphi, clo, chi, oh, w1, w2, ypool, ycur)


def _att_kernel(c0lo_ref, c0hi_ref, c1lo_ref, c1hi_ref, aoh_ref, watt_ref,
                y0_ref, y1_ref,
                cmsg_ref, ymsg_ref, aemb_ref):
    i = pl.program_id(0)
    scale = 1.0 / np.sqrt(EMB)
    watt = watt_ref[...]
    e0 = jnp.concatenate([c0lo_ref[...], c0hi_ref[...]], axis=1)
    e1 = jnp.concatenate([c1lo_ref[...], c1hi_ref[...]], axis=1)
    a0 = jnp.dot(e0, watt, preferred_element_type=jnp.float32)
    a1 = jnp.dot(e1, watt, preferred_element_type=jnp.float32)
    aoh = aoh_ref[...]
    for l, el in ((0, e0), (1, e1)):
        s0 = jnp.sum(el * a0, axis=1, keepdims=True) * scale
        s1 = jnp.sum(el * a1, axis=1, keepdims=True) * scale
        m = jnp.maximum(s0, s1)
        x0 = jnp.exp(s0 - m)
        x1 = jnp.exp(s1 - m)
        den = x0 + x1
        msg = (x0 / den) * e0 + (x1 / den) * e1
        cm = _norm_rows(msg)
        cmsg_ref[l] = cm
        ae = lax.dot_general(aoh, cm, (((0,), (0,)), ((), ())),
                             preferred_element_type=jnp.float32)

        @pl.when(i == 0)
        def _():
            aemb_ref[l] = ae

        @pl.when(i != 0)
        def _():
            aemb_ref[l] += ae

    @pl.when(i == 0)
    def _():
        ye0 = y0_ref[...]
        ye1 = y1_ref[...]
        ya0 = jnp.dot(ye0, watt, preferred_element_type=jnp.float32)
        ya1 = jnp.dot(ye1, watt, preferred_element_type=jnp.float32)
        for l, yel in ((0, ye0), (1, ye1)):
            s0 = jnp.sum(yel * ya0, axis=1, keepdims=True) * scale
            s1 = jnp.sum(yel * ya1, axis=1, keepdims=True) * scale
            m = jnp.maximum(s0, s1)
            x0 = jnp.exp(s0 - m)
            x1 = jnp.exp(s1 - m)
            den = x0 + x1
            ymsg = (x0 / den) * ye0 + (x1 / den) * ye1
            ymsg_ref[l] = _norm_rows(ymsg)


def _att_call(c0lo, c0hi, c1lo, c1hi, aoh, watt, y0, y1):
    return pl.pallas_call(
        _att_kernel,
        grid=(NB,),
        in_specs=[
            pl.BlockSpec((BN, H), lambda i: (i, 0)),
            pl.BlockSpec((BN, H), lambda i: (i, 0)),
            pl.BlockSpec((BN, H), lambda i: (i, 0)),
            pl.BlockSpec((BN, H), lambda i: (i, 0)),
            pl.BlockSpec((BN, B), lambda i: (i, 0)),
            pl.BlockSpec((EMB, EMB), lambda i: (0, 0)),
            pl.BlockSpec((B, EMB), lambda i: (0, 0)),
            pl.BlockSpec((B, EMB), lambda i: (0, 0)),
        ],
        out_specs=[
            pl.BlockSpec((LAY, BN, EMB), lambda i: (0, i, 0)),
            pl.BlockSpec((LAY, B, EMB), lambda i: (0, 0, 0)),
            pl.BlockSpec((LAY, B, EMB), lambda i: (0, 0, 0)),
        ],
        out_shape=[
            jax.ShapeDtypeStruct((LAY, N, EMB), jnp.float32),
            jax.ShapeDtypeStruct((LAY, B, EMB), jnp.float32),
            jax.ShapeDtypeStruct((LAY, B, EMB), jnp.float32),
        ],
    )(c0lo, c0hi, c1lo, c1hi, aoh, watt, y0, y1)


def _head_kernel(aemb_ref, ymsg_ref, aux0_ref, aux1_ref, h1_ref, h2p_ref,
                 crossp_ref, wl1_ref, wl2p_ref, q_ref):
    h1 = h1_ref[...]
    h2 = h2p_ref[...]
    crossp = crossp_ref[...]
    wl1 = wl1_ref[...]
    wl2 = wl2p_ref[...]
    auxs = (aux0_ref[...], aux1_ref[...])
    qs = []
    ws = []
    for l in range(LAY):
        ym = ymsg_ref[l]
        s = jnp.dot(ym, crossp, preferred_element_type=jnp.float32)[:, 0:1]
        esa = aemb_ref[l] * s
        hid = jax.nn.relu(jnp.dot(esa, h1, preferred_element_type=jnp.float32))
        q_l = (jnp.dot(hid, h2[0:RH, :], preferred_element_type=jnp.float32)
               + jnp.dot(auxs[l], h2[RH:RH + AUX, :],
                         preferred_element_type=jnp.float32))[:, 0:1]
        qs.append(q_l)
        wl = jnp.dot(jax.nn.relu(jnp.dot(ym, wl1, preferred_element_type=jnp.float32)),
                     wl2, preferred_element_type=jnp.float32)[:, 0:1]
        ws.append(wl)
    m = jnp.maximum(ws[0], ws[1])
    x0 = jnp.exp(ws[0] - m)
    x1 = jnp.exp(ws[1] - m)
    den = x0 + x1
    q_ref[...] = (x0 / den) * qs[0] + (x1 / den) * qs[1]


def _head_call(aemb, ymsg, aux0, aux1, h1, h2p, crossp, wl1, wl2p):
    return pl.pallas_call(
        _head_kernel,
        out_shape=jax.ShapeDtypeStruct((B, 1), jnp.float32),
    )(aemb, ymsg, aux0, aux1, h1, h2p, crossp, wl1, wl2p)


def kernel(edge_index, graph_ids, action_nodes, aux_input, w_n2l, p_node_conv,
           p_node_conv2, p_node_conv3, h1_weight, h2_weight, cross_product,
           w_layer1, w_layer2, W_att):
    f32 = jnp.float32
    src0 = edge_index[0, 0]
    dst0 = edge_index[0, 1]
    src1 = edge_index[1, 0]
    dst1 = edge_index[1, 1]

    oh = (graph_ids[:, None] == jnp.arange(B, dtype=graph_ids.dtype)[None, :]).astype(f32)
    aoh = (jnp.arange(N, dtype=action_nodes.dtype)[:, None] == action_nodes[None, :]).astype(f32)
    w1 = p_node_conv @ p_node_conv3[:EMB]
    w2 = p_node_conv2 @ p_node_conv3[EMB:]
    wpad = jnp.zeros((8, EMB), f32).at[0:2].set(w_n2l)
    h2p = jnp.zeros((40, 8), f32).at[:RH + AUX, 0:1].set(h2_weight)
    crossp = jnp.zeros((EMB, 8), f32).at[:, 0:1].set(cross_product)
    wl2p = jnp.zeros((128, 8), f32).at[:, 0:1].set(w_layer2)
    aux0 = aux_input[:, 0, :]
    aux1 = aux_input[:, 1, :]

    hist = _hist_call(src0, src1)
    c0lo, c0hi, ypool0, ycur0 = _prep_call(hist, oh, wpad)

    curs = []
    ycurs = []
    for l, (srcl, dstl) in enumerate(((src0, dst0), (src1, dst1))):
        clo = c0lo[l]
        chi = c0hi[l]
        ypool = ypool0[l]
        ycur = ycur0
        for _ in range(BP):
            nplo, nphi = _spmm_call(dstl, srcl, clo, chi)
            clo, chi, ypool, ycur = _dense_call(nplo, nphi, clo, chi, oh, w1, w2,
                                                ypool, ycur)
        curs.append((clo, chi))
        ycurs.append(ycur)

    cur_msg, ymsg, aemb = _att_call(curs[0][0], curs[0][1], curs[1][0], curs[1][1],
                                    aoh, W_att, ycurs[0], ycurs[1])
    q = _head_call(aemb, ymsg, aux0, aux1, h1_weight, h2p, crossp, w_layer1, wl2p)
    return (q, cur_msg)


# speedup vs baseline: 7.7253x; 1.3555x over previous
import functools

import jax
import jax.numpy as jnp
import numpy as np
from jax import lax
from jax.experimental import pallas as pl
from jax.experimental.pallas import tpu as pltpu
from jax.experimental.pallas import tpu_sc as plsc

N = 50000
B = 64
E = 800000
EMB = 64
AUX = 4
RH = 32
LAY = 2
BP = 3

H = 32
NC = 2
NS = 16
CH = 1000
CHS = 400
RPT = N // NS
EPT = E // NS
EPT2 = E // (NC * NS)
BN = 5000
NB = N // BN

_EPS = 1e-12

_sc_mesh = plsc.VectorSubcoreMesh(core_axis_name="c", subcore_axis_name="s")
_sc_params = pltpu.CompilerParams(use_tc_tiling_on_sc=False,
                                  internal_scratch_in_bytes=0)


def _zero_rows(buf, nrows, ncols):
    zv = jnp.zeros((16,), jnp.float32)

    def body(i, _):
        for j in range(ncols // 16):
            buf[i, pl.ds(j * 16, 16)] = zv
        return 0

    lax.fori_loop(0, nrows, body, 0)


def _fill_ones(buf, nrows, ncols):
    ov = jnp.ones((16,), jnp.float32)

    def body(i, _):
        for j in range(ncols // 16):
            buf[i, pl.ds(j * 16, 16)] = ov
        return 0

    lax.fori_loop(0, nrows, body, 0)


def _hist_body(src0, src1, out, ones_v, idx_v, acc):
    cid = lax.axis_index("c")
    sid = lax.axis_index("s")
    srcs = [src0, src1]
    r0 = sid * RPT
    wid = cid * NS + sid
    for l in range(LAY):
        _zero_rows(ones_v, CH, 16)
        for k in range(4):
            sz = CH if k < 3 else RPT - 3 * CH
            pltpu.sync_copy(ones_v.at[pl.ds(0, sz)], acc.at[pl.ds(r0 + k * CH, sz)])
        _fill_ones(ones_v, CH, 16)
        plsc.subcore_barrier()

        def chunk(ci, _):
            base = wid * EPT2 + ci * CH
            pltpu.sync_copy(srcs[l].at[pl.ds(base, CH)], idx_v)
            pltpu.sync_copy(ones_v, acc.at[idx_v], add=True)
            return 0

        lax.fori_loop(0, EPT2 // CH, chunk, 0)
        plsc.subcore_barrier()
        pltpu.sync_copy(acc.at[pl.ds(r0, RPT)], out.at[l, cid, pl.ds(r0, RPT)])
        plsc.subcore_barrier()


_hist_call = pl.kernel(
    _hist_body,
    out_type=jax.ShapeDtypeStruct((LAY, NC, N, 16), jnp.float32),
    mesh=_sc_mesh,
    compiler_params=_sc_params,
    scratch_types=[
        pltpu.VMEM((CH, 16), jnp.float32),
        pltpu.VMEM((CH,), jnp.int32),
        pltpu.VMEM_SHARED((N, 16), jnp.float32),
    ],
)


def _spmm_body(dst, src, tlo, thi, outlo, outhi,
               idx_d0, idx_s0, rows0, idx_d1, idx_s1, rows1,
               sem_g0, sem_g1, sem_s0, sem_s1, acc):
    cid = lax.axis_index("c")
    sid = lax.axis_index("s")
    bufs = ((idx_d0, idx_s0, rows0, sem_g0, sem_s0),
            (idx_d1, idx_s1, rows1, sem_g1, sem_s1))
    _zero_rows(rows0, CHS, H)
    r0 = sid * RPT
    nz = RPT // CHS
    for k in range(nz + 1):
        sz = CHS if k < nz else RPT - nz * CHS
        pltpu.sync_copy(rows0.at[pl.ds(0, sz)], acc.at[pl.ds(r0 + k * CHS, sz)])
    plsc.subcore_barrier()

    def load_and_gather(ci, b):
        idx_d, idx_s, rows, sem_g, _ = bufs[b]
        base = sid * EPT + ci * CHS
        pltpu.sync_copy(dst.at[pl.ds(base, CHS)], idx_d)
        pltpu.sync_copy(src.at[pl.ds(base, CHS)], idx_s)

        @pl.when(cid == 0)
        def _():
            pltpu.make_async_copy(tlo.at[idx_d], rows, sem_g).start()

        @pl.when(cid == 1)
        def _():
            pltpu.make_async_copy(thi.at[idx_d], rows, sem_g).start()

    def wait_gather(b):
        idx_d, _, rows, sem_g, _ = bufs[b]

        @pl.when(cid == 0)
        def _():
            pltpu.make_async_copy(tlo.at[idx_d], rows, sem_g).wait()

        @pl.when(cid == 1)
        def _():
            pltpu.make_async_copy(thi.at[idx_d], rows, sem_g).wait()

    def start_scatter(b):
        _, idx_s, rows, _, sem_s = bufs[b]
        pltpu.async_copy(rows, acc.at[idx_s], sem_s, add=True)

    def wait_scatter(b):
        _, idx_s, rows, _, sem_s = bufs[b]
        pltpu.make_async_copy(rows, acc.at[idx_s], sem_s).wait()

    load_and_gather(0, 0)

    def pair(k, _):
        i0 = 2 * k

        @pl.when(k > 0)
        def _():
            wait_scatter(1)

        load_and_gather(i0 + 1, 1)
        wait_gather(0)
        start_scatter(0)
        wait_scatter(0)
        load_and_gather(i0 + 2, 0)
        wait_gather(1)
        start_scatter(1)
        return 0

    lax.fori_loop(0, (EPT // CHS) // 2, pair, 0)
    wait_gather(0)
    start_scatter(0)
    wait_scatter(1)
    wait_scatter(0)
    plsc.subcore_barrier()

    @pl.when(cid == 0)
    def _():
        pltpu.sync_copy(acc.at[pl.ds(r0, RPT)], outlo.at[pl.ds(r0, RPT)])

    @pl.when(cid == 1)
    def _():
        pltpu.sync_copy(acc.at[pl.ds(r0, RPT)], outhi.at[pl.ds(r0, RPT)])


_spmm_call = pl.kernel(
    _spmm_body,
    out_type=(
        jax.ShapeDtypeStruct((N, H), jnp.float32),
        jax.ShapeDtypeStruct((N, H), jnp.float32),
    ),
    mesh=_sc_mesh,
    compiler_params=_sc_params,
    scratch_types=[
        pltpu.VMEM((CHS,), jnp.int32),
        pltpu.VMEM((CHS,), jnp.int32),
        pltpu.VMEM((CHS, H), jnp.float32),
        pltpu.VMEM((CHS,), jnp.int32),
        pltpu.VMEM((CHS,), jnp.int32),
        pltpu.VMEM((CHS, H), jnp.float32),
        pltpu.SemaphoreType.DMA,
        pltpu.SemaphoreType.DMA,
        pltpu.SemaphoreType.DMA,
        pltpu.SemaphoreType.DMA,
        pltpu.VMEM_SHARED((N, H), jnp.float32),
    ],
)


def _norm_rows(z):
    n = jnp.sqrt(jnp.sum(z * z, axis=1, keepdims=True))
    return z / jnp.maximum(n, _EPS)


def _prep_kernel(hist_ref, oh_ref, wpad_ref, c0lo_ref, c0hi_ref, ypool0_ref, ycur0_ref):
    i = pl.program_id(0)
    w = wpad_ref[...]
    u = _norm_rows(jax.nn.relu(w[0:1, :] + w[1:2, :]))
    oh = oh_ref[...]
    ones_row = jnp.ones((1, EMB), jnp.float32)
    for l in range(LAY):
        d = hist_ref[l, 0] + hist_ref[l, 1]
        dsum = jnp.sum(d, axis=1, keepdims=True)
        mask = (dsum > 0).astype(jnp.float32)
        cur0 = mask * u
        c0lo_ref[l] = cur0[:, :H]
        c0hi_ref[l] = cur0[:, H:]
        mask64 = mask * ones_row
        cnt = lax.dot_general(oh, mask64, (((0,), (0,)), ((), ())),
                              preferred_element_type=jnp.float32)

        @pl.when(i == 0)
        def _():
            ypool0_ref[l] = cnt * u

        @pl.when(i != 0)
        def _():
            ypool0_ref[l] += cnt * u

    @pl.when(i == 0)
    def _():
        ycur0_ref[...] = jnp.ones((B, 1), jnp.float32) * u


def _prep_call(hist, oh, wpad):
    return pl.pallas_call(
        _prep_kernel,
        grid=(NB,),
        in_specs=[
            pl.BlockSpec((LAY, NC, BN, 16), lambda i: (0, 0, i, 0)),
            pl.BlockSpec((BN, EMB), lambda i: (i, 0)),
            pl.BlockSpec((8, EMB), lambda i: (0, 0)),
        ],
        out_specs=[
            pl.BlockSpec((LAY, BN, H), lambda i: (0, i, 0)),
            pl.BlockSpec((LAY, BN, H), lambda i: (0, i, 0)),
            pl.BlockSpec((LAY, B, EMB), lambda i: (0, 0, 0)),
            pl.BlockSpec((B, EMB), lambda i: (0, 0)),
        ],
        out_shape=[
            jax.ShapeDtypeStruct((LAY, N, H), jnp.float32),
            jax.ShapeDtypeStruct((LAY, N, H), jnp.float32),
            jax.ShapeDtypeStruct((LAY, B, EMB), jnp.float32),
            jax.ShapeDtypeStruct((B, EMB), jnp.float32),
        ],
    )(hist, oh, wpad)


def _dense_kernel(nplo_ref, nphi_ref, clo_ref, chi_ref, oh_ref, w1_ref, w2_ref,
                  ypool_ref, ycur_ref,
                  nlo_ref, nhi_ref, ypooln_ref, ycurn_ref):
    i = pl.program_id(0)
    w1 = w1_ref[...]
    w2 = w2_ref[...]
    np64 = jnp.concatenate([nplo_ref[...], nphi_ref[...]], axis=1)
    cur64 = jnp.concatenate([clo_ref[...], chi_ref[...]], axis=1)
    z = jax.nn.relu(
        jnp.dot(np64, w1, preferred_element_type=jnp.float32)
        + jnp.dot(cur64, w2, preferred_element_type=jnp.float32))
    new = _norm_rows(z)
    nlo_ref[...] = new[:, :H]
    nhi_ref[...] = new[:, H:]
    ypn = lax.dot_general(oh_ref[...], new, (((0,), (0,)), ((), ())),
                          preferred_element_type=jnp.float32)

    @pl.when(i == 0)
    def _():
        ypooln_ref[...] = ypn
        yz = jax.nn.relu(
            jnp.dot(ypool_ref[...], w1, preferred_element_type=jnp.float32)
            + jnp.dot(ycur_ref[...], w2, preferred_element_type=jnp.float32))
        ycurn_ref[...] = _norm_rows(yz)

    @pl.when(i != 0)
    def _():
        ypooln_ref[...] += ypn


def _dense_call(nplo, nphi, clo, chi, oh, w1, w2, ypool, ycur):
    return pl.pallas_call(
        _dense_kernel,
        grid=(NB,),
        in_specs=[
            pl.BlockSpec((BN, H), lambda i: (i, 0)),
            pl.BlockSpec((BN, H), lambda i: (i, 0)),
            pl.BlockSpec((BN, H), lambda i: (i, 0)),
            pl.BlockSpec((BN, H), lambda i: (i, 0)),
            pl.BlockSpec((BN, EMB), lambda i: (i, 0)),
            pl.BlockSpec((EMB, EMB), lambda i: (0, 0)),
            pl.BlockSpec((EMB, EMB), lambda i: (0, 0)),
            pl.BlockSpec((B, EMB), lambda i: (0, 0)),
            pl.BlockSpec((B, EMB), lambda i: (0, 0)),
        ],
        out_specs=[
            pl.BlockSpec((BN, H), lambda i: (i, 0)),
            pl.BlockSpec((BN, H), lambda i: (i, 0)),
            pl.BlockSpec((B, EMB), lambda i: (0, 0)),
            pl.BlockSpec((B, EMB), lambda i: (0, 0)),
        ],
        out_shape=[
            jax.ShapeDtypeStruct((N, H), jnp.float32),
            jax.ShapeDtypeStruct((N, H), jnp.float32),
            jax.ShapeDtypeStruct((B, EMB), jnp.float32),
            jax.ShapeDtypeStruct((B, EMB), jnp.float32),
        ],
    )(nplo, nphi, clo, chi, oh, w1, w2, ypool, ycur)


def _att_kernel(c0lo_ref, c0hi_ref, c1lo_ref, c1hi_ref, aoh_ref, watt_ref,
                y0_ref, y1_ref,
                cmsg_ref, ymsg_ref, aemb_ref):
    i = pl.program_id(0)
    scale = 1.0 / np.sqrt(EMB)
    watt = watt_ref[...]
    e0 = jnp.concatenate([c0lo_ref[...], c0hi_ref[...]], axis=1)
    e1 = jnp.concatenate([c1lo_ref[...], c1hi_ref[...]], axis=1)
    a0 = jnp.dot(e0, watt, preferred_element_type=jnp.float32)
    a1 = jnp.dot(e1, watt, preferred_element_type=jnp.float32)
    aoh = aoh_ref[...]
    for l, el in ((0, e0), (1, e1)):
        s0 = jnp.sum(el * a0, axis=1, keepdims=True) * scale
        s1 = jnp.sum(el * a1, axis=1, keepdims=True) * scale
        m = jnp.maximum(s0, s1)
        x0 = jnp.exp(s0 - m)
        x1 = jnp.exp(s1 - m)
        den = x0 + x1
        msg = (x0 / den) * e0 + (x1 / den) * e1
        cm = _norm_rows(msg)
        cmsg_ref[l] = cm
        ae = lax.dot_general(aoh, cm, (((0,), (0,)), ((), ())),
                             preferred_element_type=jnp.float32)

        @pl.when(i == 0)
        def _():
            aemb_ref[l] = ae

        @pl.when(i != 0)
        def _():
            aemb_ref[l] += ae

    @pl.when(i == 0)
    def _():
        ye0 = y0_ref[...]
        ye1 = y1_ref[...]
        ya0 = jnp.dot(ye0, watt, preferred_element_type=jnp.float32)
        ya1 = jnp.dot(ye1, watt, preferred_element_type=jnp.float32)
        for l, yel in ((0, ye0), (1, ye1)):
            s0 = jnp.sum(yel * ya0, axis=1, keepdims=True) * scale
            s1 = jnp.sum(yel * ya1, axis=1, keepdims=True) * scale
            m = jnp.maximum(s0, s1)
            x0 = jnp.exp(s0 - m)
            x1 = jnp.exp(s1 - m)
            den = x0 + x1
            ymsg = (x0 / den) * ye0 + (x1 / den) * ye1
            ymsg_ref[l] = _norm_rows(ymsg)


def _att_call(c0lo, c0hi, c1lo, c1hi, aoh, watt, y0, y1):
    return pl.pallas_call(
        _att_kernel,
        grid=(NB,),
        in_specs=[
            pl.BlockSpec((BN, H), lambda i: (i, 0)),
            pl.BlockSpec((BN, H), lambda i: (i, 0)),
            pl.BlockSpec((BN, H), lambda i: (i, 0)),
            pl.BlockSpec((BN, H), lambda i: (i, 0)),
            pl.BlockSpec((BN, B), lambda i: (i, 0)),
            pl.BlockSpec((EMB, EMB), lambda i: (0, 0)),
            pl.BlockSpec((B, EMB), lambda i: (0, 0)),
            pl.BlockSpec((B, EMB), lambda i: (0, 0)),
        ],
        out_specs=[
            pl.BlockSpec((LAY, BN, EMB), lambda i: (0, i, 0)),
            pl.BlockSpec((LAY, B, EMB), lambda i: (0, 0, 0)),
            pl.BlockSpec((LAY, B, EMB), lambda i: (0, 0, 0)),
        ],
        out_shape=[
            jax.ShapeDtypeStruct((LAY, N, EMB), jnp.float32),
            jax.ShapeDtypeStruct((LAY, B, EMB), jnp.float32),
            jax.ShapeDtypeStruct((LAY, B, EMB), jnp.float32),
        ],
    )(c0lo, c0hi, c1lo, c1hi, aoh, watt, y0, y1)


def _head_kernel(aemb_ref, ymsg_ref, aux0_ref, aux1_ref, h1_ref, h2p_ref,
                 crossp_ref, wl1_ref, wl2p_ref, q_ref):
    h1 = h1_ref[...]
    h2 = h2p_ref[...]
    crossp = crossp_ref[...]
    wl1 = wl1_ref[...]
    wl2 = wl2p_ref[...]
    auxs = (aux0_ref[...], aux1_ref[...])
    qs = []
    ws = []
    for l in range(LAY):
        ym = ymsg_ref[l]
        s = jnp.dot(ym, crossp, preferred_element_type=jnp.float32)[:, 0:1]
        esa = aemb_ref[l] * s
        hid = jax.nn.relu(jnp.dot(esa, h1, preferred_element_type=jnp.float32))
        q_l = (jnp.dot(hid, h2[0:RH, :], preferred_element_type=jnp.float32)
               + jnp.dot(auxs[l], h2[RH:RH + AUX, :],
                         preferred_element_type=jnp.float32))[:, 0:1]
        qs.append(q_l)
        wl = jnp.dot(jax.nn.relu(jnp.dot(ym, wl1, preferred_element_type=jnp.float32)),
                     wl2, preferred_element_type=jnp.float32)[:, 0:1]
        ws.append(wl)
    m = jnp.maximum(ws[0], ws[1])
    x0 = jnp.exp(ws[0] - m)
    x1 = jnp.exp(ws[1] - m)
    den = x0 + x1
    q_ref[...] = (x0 / den) * qs[0] + (x1 / den) * qs[1]


def _head_call(aemb, ymsg, aux0, aux1, h1, h2p, crossp, wl1, wl2p):
    return pl.pallas_call(
        _head_kernel,
        out_shape=jax.ShapeDtypeStruct((B, 1), jnp.float32),
    )(aemb, ymsg, aux0, aux1, h1, h2p, crossp, wl1, wl2p)


def kernel(edge_index, graph_ids, action_nodes, aux_input, w_n2l, p_node_conv,
           p_node_conv2, p_node_conv3, h1_weight, h2_weight, cross_product,
           w_layer1, w_layer2, W_att):
    f32 = jnp.float32
    src0 = edge_index[0, 0]
    dst0 = edge_index[0, 1]
    src1 = edge_index[1, 0]
    dst1 = edge_index[1, 1]

    oh = (graph_ids[:, None] == jnp.arange(B, dtype=graph_ids.dtype)[None, :]).astype(f32)
    aoh = (jnp.arange(N, dtype=action_nodes.dtype)[:, None] == action_nodes[None, :]).astype(f32)
    w1 = p_node_conv @ p_node_conv3[:EMB]
    w2 = p_node_conv2 @ p_node_conv3[EMB:]
    wpad = jnp.zeros((8, EMB), f32).at[0:2].set(w_n2l)
    h2p = jnp.zeros((40, 8), f32).at[:RH + AUX, 0:1].set(h2_weight)
    crossp = jnp.zeros((EMB, 8), f32).at[:, 0:1].set(cross_product)
    wl2p = jnp.zeros((128, 8), f32).at[:, 0:1].set(w_layer2)
    aux0 = aux_input[:, 0, :]
    aux1 = aux_input[:, 1, :]

    hist = _hist_call(src0, src1)
    c0lo, c0hi, ypool0, ycur0 = _prep_call(hist, oh, wpad)

    curs = []
    ycurs = []
    for l, (srcl, dstl) in enumerate(((src0, dst0), (src1, dst1))):
        clo = c0lo[l]
        chi = c0hi[l]
        ypool = ypool0[l]
        ycur = ycur0
        for _ in range(BP):
            nplo, nphi = _spmm_call(dstl, srcl, clo, chi)
            clo, chi, ypool, ycur = _dense_call(nplo, nphi, clo, chi, oh, w1, w2,
                                                ypool, ycur)
        curs.append((clo, chi))
        ycurs.append(ycur)

    cur_msg, ymsg, aemb = _att_call(curs[0][0], curs[0][1], curs[1][0], curs[1][1],
                                    aoh, W_att, ycurs[0], ycurs[1])
    q = _head_call(aemb, ymsg, aux0, aux1, h1_weight, h2p, crossp, w_layer1, wl2p)
    return (q, cur_msg)


# speedup vs baseline: 9.0095x; 1.1662x over previous
import functools

import jax
import jax.numpy as jnp
import numpy as np
from jax import lax
from jax.experimental import pallas as pl
from jax.experimental.pallas import tpu as pltpu
from jax.experimental.pallas import tpu_sc as plsc

N = 50000
B = 64
E = 800000
EMB = 64
AUX = 4
RH = 32
LAY = 2
BP = 3

H = 32
NC = 2
NS = 16
CH = 1000
CHS = 400
RPT = N // NS
EPT = E // NS
EPT2 = E // (NC * NS)
BN = 5000
NB = N // BN

_EPS = 1e-12

_sc_mesh = plsc.VectorSubcoreMesh(core_axis_name="c", subcore_axis_name="s")
_sc_params = pltpu.CompilerParams(use_tc_tiling_on_sc=False,
                                  internal_scratch_in_bytes=0)


def _zero_rows(buf, nrows, ncols):
    zv = jnp.zeros((16,), jnp.float32)

    def body(i, _):
        for j in range(ncols // 16):
            buf[i, pl.ds(j * 16, 16)] = zv
        return 0

    lax.fori_loop(0, nrows, body, 0)


def _fill_ones(buf, nrows, ncols):
    ov = jnp.ones((16,), jnp.float32)

    def body(i, _):
        for j in range(ncols // 16):
            buf[i, pl.ds(j * 16, 16)] = ov
        return 0

    lax.fori_loop(0, nrows, body, 0)


def _hist_body(src0, src1, out, ones_v, idx_v, acc):
    cid = lax.axis_index("c")
    sid = lax.axis_index("s")
    srcs = [src0, src1]
    r0 = sid * RPT
    wid = cid * NS + sid
    for l in range(LAY):
        _zero_rows(ones_v, CH, 16)
        for k in range(4):
            sz = CH if k < 3 else RPT - 3 * CH
            pltpu.sync_copy(ones_v.at[pl.ds(0, sz)], acc.at[pl.ds(r0 + k * CH, sz)])
        _fill_ones(ones_v, CH, 16)
        plsc.subcore_barrier()

        def chunk(ci, _):
            base = wid * EPT2 + ci * CH
            pltpu.sync_copy(srcs[l].at[pl.ds(base, CH)], idx_v)
            pltpu.sync_copy(ones_v, acc.at[idx_v], add=True)
            return 0

        lax.fori_loop(0, EPT2 // CH, chunk, 0)
        plsc.subcore_barrier()
        pltpu.sync_copy(acc.at[pl.ds(r0, RPT)], out.at[l, cid, pl.ds(r0, RPT)])
        plsc.subcore_barrier()


_hist_call = pl.kernel(
    _hist_body,
    out_type=jax.ShapeDtypeStruct((LAY, NC, N, 16), jnp.float32),
    mesh=_sc_mesh,
    compiler_params=_sc_params,
    scratch_types=[
        pltpu.VMEM((CH, 16), jnp.float32),
        pltpu.VMEM((CH,), jnp.int32),
        pltpu.VMEM_SHARED((N, 16), jnp.float32),
    ],
)


def _spmm_body(dst, src, tlo, thi, outlo, outhi,
               idx_d0, idx_s0, rows0, idx_d1, idx_s1, rows1,
               sem_g0, sem_g1, sem_s0, sem_s1, sem_id0, sem_id1,
               sem_is0, sem_is1, acc):
    cid = lax.axis_index("c")
    sid = lax.axis_index("s")
    bufs = ((idx_d0, idx_s0, rows0, sem_g0, sem_s0, sem_id0, sem_is0),
            (idx_d1, idx_s1, rows1, sem_g1, sem_s1, sem_id1, sem_is1))
    _zero_rows(rows0, CHS, H)
    r0 = sid * RPT
    nz = RPT // CHS
    for k in range(nz + 1):
        sz = CHS if k < nz else RPT - nz * CHS
        pltpu.sync_copy(rows0.at[pl.ds(0, sz)], acc.at[pl.ds(r0 + k * CHS, sz)])
    plsc.subcore_barrier()

    def start_idx_d(ci, b):
        idx_d, _, _, _, _, sem_id, _ = bufs[b]
        base = sid * EPT + ci * CHS
        pltpu.make_async_copy(dst.at[pl.ds(base, CHS)], idx_d, sem_id).start()

    def wait_idx_d(b):
        idx_d, _, _, _, _, sem_id, _ = bufs[b]
        pltpu.make_async_copy(dst.at[pl.ds(0, CHS)], idx_d, sem_id).wait()

    def start_idx_s(ci, b):
        _, idx_s, _, _, _, _, sem_is = bufs[b]
        base = sid * EPT + ci * CHS
        pltpu.make_async_copy(src.at[pl.ds(base, CHS)], idx_s, sem_is).start()

    def wait_idx_s(b):
        _, idx_s, _, _, _, _, sem_is = bufs[b]
        pltpu.make_async_copy(src.at[pl.ds(0, CHS)], idx_s, sem_is).wait()

    def start_gather(b):
        idx_d, _, rows, sem_g, _, _, _ = bufs[b]

        @pl.when(cid == 0)
        def _():
            pltpu.make_async_copy(tlo.at[idx_d], rows, sem_g).start()

        @pl.when(cid == 1)
        def _():
            pltpu.make_async_copy(thi.at[idx_d], rows, sem_g).start()

    def wait_gather(b):
        idx_d, _, rows, sem_g, _, _, _ = bufs[b]

        @pl.when(cid == 0)
        def _():
            pltpu.make_async_copy(tlo.at[idx_d], rows, sem_g).wait()

        @pl.when(cid == 1)
        def _():
            pltpu.make_async_copy(thi.at[idx_d], rows, sem_g).wait()

    def start_scatter(b):
        _, idx_s, rows, _, sem_s, _, _ = bufs[b]
        pltpu.async_copy(rows, acc.at[idx_s], sem_s, add=True)

    def wait_scatter(b):
        _, idx_s, rows, _, sem_s, _, _ = bufs[b]
        pltpu.make_async_copy(rows, acc.at[idx_s], sem_s).wait()

    NCHK = EPT // CHS
    NPAIR = NCHK // 2
    base0 = sid * EPT
    pltpu.sync_copy(dst.at[pl.ds(base0, CHS)], idx_d0)
    pltpu.sync_copy(src.at[pl.ds(base0, CHS)], idx_s0)
    start_gather(0)
    pltpu.sync_copy(dst.at[pl.ds(base0 + CHS, CHS)], idx_d1)
    pltpu.sync_copy(src.at[pl.ds(base0 + CHS, CHS)], idx_s1)
    start_gather(1)

    def pair(k, _):
        i0 = 2 * k
        wait_gather(0)
        start_idx_d(i0 + 2, 0)

        @pl.when(k > 0)
        def _():
            wait_idx_s(0)

        start_scatter(0)
        wait_gather(1)

        @pl.when(k < NPAIR - 1)
        def _():
            start_idx_d(i0 + 3, 1)

        @pl.when(k > 0)
        def _():
            wait_idx_s(1)

        start_scatter(1)
        wait_scatter(0)
        start_idx_s(i0 + 2, 0)
        wait_idx_d(0)
        start_gather(0)

        @pl.when(k < NPAIR - 1)
        def _():
            wait_scatter(1)
            start_idx_s(i0 + 3, 1)
            wait_idx_d(1)
            start_gather(1)

        return 0

    lax.fori_loop(0, NPAIR, pair, 0)
    wait_gather(0)
    wait_idx_s(0)
    start_scatter(0)
    wait_scatter(1)
    wait_scatter(0)
    plsc.subcore_barrier()

    @pl.when(cid == 0)
    def _():
        pltpu.sync_copy(acc.at[pl.ds(r0, RPT)], outlo.at[pl.ds(r0, RPT)])

    @pl.when(cid == 1)
    def _():
        pltpu.sync_copy(acc.at[pl.ds(r0, RPT)], outhi.at[pl.ds(r0, RPT)])


_spmm_call = pl.kernel(
    _spmm_body,
    out_type=(
        jax.ShapeDtypeStruct((N, H), jnp.float32),
        jax.ShapeDtypeStruct((N, H), jnp.float32),
    ),
    mesh=_sc_mesh,
    compiler_params=_sc_params,
    scratch_types=[
        pltpu.VMEM((CHS,), jnp.int32),
        pltpu.VMEM((CHS,), jnp.int32),
        pltpu.VMEM((CHS, H), jnp.float32),
        pltpu.VMEM((CHS,), jnp.int32),
        pltpu.VMEM((CHS,), jnp.int32),
        pltpu.VMEM((CHS, H), jnp.float32),
        pltpu.SemaphoreType.DMA,
        pltpu.SemaphoreType.DMA,
        pltpu.SemaphoreType.DMA,
        pltpu.SemaphoreType.DMA,
        pltpu.SemaphoreType.DMA,
        pltpu.SemaphoreType.DMA,
        pltpu.SemaphoreType.DMA,
        pltpu.SemaphoreType.DMA,
        pltpu.VMEM_SHARED((N, H), jnp.float32),
    ],
)


def _norm_rows(z):
    n = jnp.sqrt(jnp.sum(z * z, axis=1, keepdims=True))
    return z / jnp.maximum(n, _EPS)


def _prep_kernel(hist_ref, oh_ref, wpad_ref, c0lo_ref, c0hi_ref, ypool0_ref, ycur0_ref):
    i = pl.program_id(0)
    w = wpad_ref[...]
    u = _norm_rows(jax.nn.relu(w[0:1, :] + w[1:2, :]))
    oh = oh_ref[...]
    ones_row = jnp.ones((1, EMB), jnp.float32)
    for l in range(LAY):
        d = hist_ref[l, 0] + hist_ref[l, 1]
        dsum = jnp.sum(d, axis=1, keepdims=True)
        mask = (dsum > 0).astype(jnp.float32)
        cur0 = mask * u
        c0lo_ref[l] = cur0[:, :H]
        c0hi_ref[l] = cur0[:, H:]
        mask64 = mask * ones_row
        cnt = lax.dot_general(oh, mask64, (((0,), (0,)), ((), ())),
                              preferred_element_type=jnp.float32)

        @pl.when(i == 0)
        def _():
            ypool0_ref[l] = cnt * u

        @pl.when(i != 0)
        def _():
            ypool0_ref[l] += cnt * u

    @pl.when(i == 0)
    def _():
        ycur0_ref[...] = jnp.ones((B, 1), jnp.float32) * u


def _prep_call(hist, oh, wpad):
    return pl.pallas_call(
        _prep_kernel,
        grid=(NB,),
        in_specs=[
            pl.BlockSpec((LAY, NC, BN, 16), lambda i: (0, 0, i, 0)),
            pl.BlockSpec((BN, EMB), lambda i: (i, 0)),
            pl.BlockSpec((8, EMB), lambda i: (0, 0)),
        ],
        out_specs=[
            pl.BlockSpec((LAY, BN, H), lambda i: (0, i, 0)),
            pl.BlockSpec((LAY, BN, H), lambda i: (0, i, 0)),
            pl.BlockSpec((LAY, B, EMB), lambda i: (0, 0, 0)),
            pl.BlockSpec((B, EMB), lambda i: (0, 0)),
        ],
        out_shape=[
            jax.ShapeDtypeStruct((LAY, N, H), jnp.float32),
            jax.ShapeDtypeStruct((LAY, N, H), jnp.float32),
            jax.ShapeDtypeStruct((LAY, B, EMB), jnp.float32),
            jax.ShapeDtypeStruct((B, EMB), jnp.float32),
        ],
    )(hist, oh, wpad)


def _dense_kernel(nplo_ref, nphi_ref, clo_ref, chi_ref, oh_ref, w1_ref, w2_ref,
                  ypool_ref, ycur_ref,
                  nlo_ref, nhi_ref, ypooln_ref, ycurn_ref):
    i = pl.program_id(0)
    w1 = w1_ref[...]
    w2 = w2_ref[...]
    np64 = jnp.concatenate([nplo_ref[...], nphi_ref[...]], axis=1)
    cur64 = jnp.concatenate([clo_ref[...], chi_ref[...]], axis=1)
    z = jax.nn.relu(
        jnp.dot(np64, w1, preferred_element_type=jnp.float32)
        + jnp.dot(cur64, w2, preferred_element_type=jnp.float32))
    new = _norm_rows(z)
    nlo_ref[...] = new[:, :H]
    nhi_ref[...] = new[:, H:]
    ypn = lax.dot_general(oh_ref[...], new, (((0,), (0,)), ((), ())),
                          preferred_element_type=jnp.float32)

    @pl.when(i == 0)
    def _():
        ypooln_ref[...] = ypn
        yz = jax.nn.relu(
            jnp.dot(ypool_ref[...], w1, preferred_element_type=jnp.float32)
            + jnp.dot(ycur_ref[...], w2, preferred_element_type=jnp.float32))
        ycurn_ref[...] = _norm_rows(yz)

    @pl.when(i != 0)
    def _():
        ypooln_ref[...] += ypn


def _dense_call(nplo, nphi, clo, chi, oh, w1, w2, ypool, ycur):
    return pl.pallas_call(
        _dense_kernel,
        grid=(NB,),
        in_specs=[
            pl.BlockSpec((BN, H), lambda i: (i, 0)),
            pl.BlockSpec((BN, H), lambda i: (i, 0)),
            pl.BlockSpec((BN, H), lambda i: (i, 0)),
            pl.BlockSpec((BN, H), lambda i: (i, 0)),
            pl.BlockSpec((BN, EMB), lambda i: (i, 0)),
            pl.BlockSpec((EMB, EMB), lambda i: (0, 0)),
            pl.BlockSpec((EMB, EMB), lambda i: (0, 0)),
            pl.BlockSpec((B, EMB), lambda i: (0, 0)),
            pl.BlockSpec((B, EMB), lambda i: (0, 0)),
        ],
        out_specs=[
            pl.BlockSpec((BN, H), lambda i: (i, 0)),
            pl.BlockSpec((BN, H), lambda i: (i, 0)),
            pl.BlockSpec((B, EMB), lambda i: (0, 0)),
            pl.BlockSpec((B, EMB), lambda i: (0, 0)),
        ],
        out_shape=[
            jax.ShapeDtypeStruct((N, H), jnp.float32),
            jax.ShapeDtypeStruct((N, H), jnp.float32),
            jax.ShapeDtypeStruct((B, EMB), jnp.float32),
            jax.ShapeDtypeStruct((B, EMB), jnp.float32),
        ],
    )(nplo, nphi, clo, chi, oh, w1, w2, ypool, ycur)


def _att_kernel(c0lo_ref, c0hi_ref, c1lo_ref, c1hi_ref, aoh_ref, watt_ref,
                y0_ref, y1_ref,
                cmsg_ref, ymsg_ref, aemb_ref):
    i = pl.program_id(0)
    scale = 1.0 / np.sqrt(EMB)
    watt = watt_ref[...]
    e0 = jnp.concatenate([c0lo_ref[...], c0hi_ref[...]], axis=1)
    e1 = jnp.concatenate([c1lo_ref[...], c1hi_ref[...]], axis=1)
    a0 = jnp.dot(e0, watt, preferred_element_type=jnp.float32)
    a1 = jnp.dot(e1, watt, preferred_element_type=jnp.float32)
    aoh = aoh_ref[...]
    for l, el in ((0, e0), (1, e1)):
        s0 = jnp.sum(el * a0, axis=1, keepdims=True) * scale
        s1 = jnp.sum(el * a1, axis=1, keepdims=True) * scale
        m = jnp.maximum(s0, s1)
        x0 = jnp.exp(s0 - m)
        x1 = jnp.exp(s1 - m)
        den = x0 + x1
        msg = (x0 / den) * e0 + (x1 / den) * e1
        cm = _norm_rows(msg)
        cmsg_ref[l] = cm
        ae = lax.dot_general(aoh, cm, (((0,), (0,)), ((), ())),
                             preferred_element_type=jnp.float32)

        @pl.when(i == 0)
        def _():
            aemb_ref[l] = ae

        @pl.when(i != 0)
        def _():
            aemb_ref[l] += ae

    @pl.when(i == 0)
    def _():
        ye0 = y0_ref[...]
        ye1 = y1_ref[...]
        ya0 = jnp.dot(ye0, watt, preferred_element_type=jnp.float32)
        ya1 = jnp.dot(ye1, watt, preferred_element_type=jnp.float32)
        for l, yel in ((0, ye0), (1, ye1)):
            s0 = jnp.sum(yel * ya0, axis=1, keepdims=True) * scale
            s1 = jnp.sum(yel * ya1, axis=1, keepdims=True) * scale
            m = jnp.maximum(s0, s1)
            x0 = jnp.exp(s0 - m)
            x1 = jnp.exp(s1 - m)
            den = x0 + x1
            ymsg = (x0 / den) * ye0 + (x1 / den) * ye1
            ymsg_ref[l] = _norm_rows(ymsg)


def _att_call(c0lo, c0hi, c1lo, c1hi, aoh, watt, y0, y1):
    return pl.pallas_call(
        _att_kernel,
        grid=(NB,),
        in_specs=[
            pl.BlockSpec((BN, H), lambda i: (i, 0)),
            pl.BlockSpec((BN, H), lambda i: (i, 0)),
            pl.BlockSpec((BN, H), lambda i: (i, 0)),
            pl.BlockSpec((BN, H), lambda i: (i, 0)),
            pl.BlockSpec((BN, B), lambda i: (i, 0)),
            pl.BlockSpec((EMB, EMB), lambda i: (0, 0)),
            pl.BlockSpec((B, EMB), lambda i: (0, 0)),
            pl.BlockSpec((B, EMB), lambda i: (0, 0)),
        ],
        out_specs=[
            pl.BlockSpec((LAY, BN, EMB), lambda i: (0, i, 0)),
            pl.BlockSpec((LAY, B, EMB), lambda i: (0, 0, 0)),
            pl.BlockSpec((LAY, B, EMB), lambda i: (0, 0, 0)),
        ],
        out_shape=[
            jax.ShapeDtypeStruct((LAY, N, EMB), jnp.float32),
            jax.ShapeDtypeStruct((LAY, B, EMB), jnp.float32),
            jax.ShapeDtypeStruct((LAY, B, EMB), jnp.float32),
        ],
    )(c0lo, c0hi, c1lo, c1hi, aoh, watt, y0, y1)


def _head_kernel(aemb_ref, ymsg_ref, aux0_ref, aux1_ref, h1_ref, h2p_ref,
                 crossp_ref, wl1_ref, wl2p_ref, q_ref):
    h1 = h1_ref[...]
    h2 = h2p_ref[...]
    crossp = crossp_ref[...]
    wl1 = wl1_ref[...]
    wl2 = wl2p_ref[...]
    auxs = (aux0_ref[...], aux1_ref[...])
    qs = []
    ws = []
    for l in range(LAY):
        ym = ymsg_ref[l]
        s = jnp.dot(ym, crossp, preferred_element_type=jnp.float32)[:, 0:1]
        esa = aemb_ref[l] * s
        hid = jax.nn.relu(jnp.dot(esa, h1, preferred_element_type=jnp.float32))
        q_l = (jnp.dot(hid, h2[0:RH, :], preferred_element_type=jnp.float32)
               + jnp.dot(auxs[l], h2[RH:RH + AUX, :],
                         preferred_element_type=jnp.float32))[:, 0:1]
        qs.append(q_l)
        wl = jnp.dot(jax.nn.relu(jnp.dot(ym, wl1, preferred_element_type=jnp.float32)),
                     wl2, preferred_element_type=jnp.float32)[:, 0:1]
        ws.append(wl)
    m = jnp.maximum(ws[0], ws[1])
    x0 = jnp.exp(ws[0] - m)
    x1 = jnp.exp(ws[1] - m)
    den = x0 + x1
    q_ref[...] = (x0 / den) * qs[0] + (x1 / den) * qs[1]


def _head_call(aemb, ymsg, aux0, aux1, h1, h2p, crossp, wl1, wl2p):
    return pl.pallas_call(
        _head_kernel,
        out_shape=jax.ShapeDtypeStruct((B, 1), jnp.float32),
    )(aemb, ymsg, aux0, aux1, h1, h2p, crossp, wl1, wl2p)


def kernel(edge_index, graph_ids, action_nodes, aux_input, w_n2l, p_node_conv,
           p_node_conv2, p_node_conv3, h1_weight, h2_weight, cross_product,
           w_layer1, w_layer2, W_att):
    f32 = jnp.float32
    src0 = edge_index[0, 0]
    dst0 = edge_index[0, 1]
    src1 = edge_index[1, 0]
    dst1 = edge_index[1, 1]

    oh = (graph_ids[:, None] == jnp.arange(B, dtype=graph_ids.dtype)[None, :]).astype(f32)
    aoh = (jnp.arange(N, dtype=action_nodes.dtype)[:, None] == action_nodes[None, :]).astype(f32)
    w1 = p_node_conv @ p_node_conv3[:EMB]
    w2 = p_node_conv2 @ p_node_conv3[EMB:]
    wpad = jnp.zeros((8, EMB), f32).at[0:2].set(w_n2l)
    h2p = jnp.zeros((40, 8), f32).at[:RH + AUX, 0:1].set(h2_weight)
    crossp = jnp.zeros((EMB, 8), f32).at[:, 0:1].set(cross_product)
    wl2p = jnp.zeros((128, 8), f32).at[:, 0:1].set(w_layer2)
    aux0 = aux_input[:, 0, :]
    aux1 = aux_input[:, 1, :]

    hist = _hist_call(src0, src1)
    c0lo, c0hi, ypool0, ycur0 = _prep_call(hist, oh, wpad)

    curs = []
    ycurs = []
    for l, (srcl, dstl) in enumerate(((src0, dst0), (src1, dst1))):
        clo = c0lo[l]
        chi = c0hi[l]
        ypool = ypool0[l]
        ycur = ycur0
        for _ in range(BP):
            nplo, nphi = _spmm_call(dstl, srcl, clo, chi)
            clo, chi, ypool, ycur = _dense_call(nplo, nphi, clo, chi, oh, w1, w2,
                                                ypool, ycur)
        curs.append((clo, chi))
        ycurs.append(ycur)

    cur_msg, ymsg, aemb = _att_call(curs[0][0], curs[0][1], curs[1][0], curs[1][1],
                                    aoh, W_att, ycurs[0], ycurs[1])
    q = _head_call(aemb, ymsg, aux0, aux1, h1_weight, h2p, crossp, w_layer1, wl2p)
    return (q, cur_msg)


# speedup vs baseline: 9.8272x; 1.0908x over previous
import functools

import jax
import jax.numpy as jnp
import numpy as np
from jax import lax
from jax.experimental import pallas as pl
from jax.experimental.pallas import tpu as pltpu
from jax.experimental.pallas import tpu_sc as plsc

N = 50000
B = 64
E = 800000
EMB = 64
AUX = 4
RH = 32
LAY = 2
BP = 3

H = 32
NC = 2
NS = 16
CH = 1000
CHS = 400
RPT = N // NS
EPT = E // NS
EPT2 = E // (NC * NS)

NP = 51200
NP4 = NP // 4
NP8 = NP // 8
BN4 = 1280
BN8 = 640
NB = NP4 // BN4

_EPS = 1e-12

_sc_mesh = plsc.VectorSubcoreMesh(core_axis_name="c", subcore_axis_name="s")
_sc_params = pltpu.CompilerParams(use_tc_tiling_on_sc=False,
                                  internal_scratch_in_bytes=0)


def _zero_rows(buf, nrows, ncols):
    zv = jnp.zeros((16,), jnp.float32)

    def body(i, _):
        for j in range(ncols // 16):
            buf[i, pl.ds(j * 16, 16)] = zv
        return 0

    lax.fori_loop(0, nrows, body, 0)


def _fill_ones(buf, nrows, ncols):
    ov = jnp.ones((16,), jnp.float32)

    def body(i, _):
        for j in range(ncols // 16):
            buf[i, pl.ds(j * 16, 16)] = ov
        return 0

    lax.fori_loop(0, nrows, body, 0)


def _g32():
    r = lax.broadcasted_iota(jnp.int32, (128, 4), 0) // H
    c = lax.broadcasted_iota(jnp.int32, (128, 4), 1)
    return (r == c).astype(jnp.float32)


def _g32t():
    r = lax.broadcasted_iota(jnp.int32, (4, 128), 0)
    c = lax.broadcasted_iota(jnp.int32, (4, 128), 1) // H
    return (r == c).astype(jnp.float32)


def _mm(a, b):
    return jnp.dot(a, b, preferred_element_type=jnp.float32)


def _dot0(a, b):
    return lax.dot_general(a, b, (((0,), (0,)), ((), ())),
                           preferred_element_type=jnp.float32)


def _norm_rows(z):
    n = jnp.sqrt(jnp.sum(z * z, axis=1, keepdims=True))
    return z / jnp.maximum(n, _EPS)


def _hist_body(src0, src1, h00, h01, h10, h11, idx_v, ones_v, acc):
    cid = lax.axis_index("c")
    sid = lax.axis_index("s")
    srcs = [src0, src1]
    houts = [[h00, h01], [h10, h11]]
    r0 = sid * RPT
    wid = cid * NS + sid
    for l in range(LAY):
        _zero_rows(ones_v, CH, 16)
        for k in range(4):
            sz = CH if k < 3 else RPT - 3 * CH
            pltpu.sync_copy(ones_v.at[pl.ds(0, sz)], acc.at[pl.ds(r0 + k * CH, sz)])
        _fill_ones(ones_v, CH, 16)
        plsc.subcore_barrier()

        def chunk(ci, _):
            base = wid * EPT2 + ci * CH
            pltpu.sync_copy(srcs[l].at[pl.ds(base, CH)], idx_v)
            pltpu.sync_copy(ones_v, acc.at[idx_v], add=True)
            return 0

        lax.fori_loop(0, EPT2 // CH, chunk, 0)
        plsc.subcore_barrier()

        @pl.when(cid == 0)
        def _():
            pltpu.sync_copy(acc.at[pl.ds(r0, RPT)], houts[l][0].at[pl.ds(r0, RPT)])

        @pl.when(cid == 1)
        def _():
            pltpu.sync_copy(acc.at[pl.ds(r0, RPT)], houts[l][1].at[pl.ds(r0, RPT)])

        plsc.subcore_barrier()


_hist_call = pl.kernel(
    _hist_body,
    out_type=tuple(jax.ShapeDtypeStruct((NP, 16), jnp.float32) for _ in range(4)),
    mesh=_sc_mesh,
    compiler_params=_sc_params,
    scratch_types=[
        pltpu.VMEM((CH,), jnp.int32),
        pltpu.VMEM((CH, 16), jnp.float32),
        pltpu.VMEM_SHARED((N, 16), jnp.float32),
    ],
)


def _spmm_body(dst, src, tlo, thi, outlo, outhi,
               idx_d0, idx_s0, rows0, idx_d1, idx_s1, rows1,
               sem_g0, sem_g1, sem_s0, sem_s1, sem_id0, sem_id1,
               sem_is0, sem_is1, acc):
    cid = lax.axis_index("c")
    sid = lax.axis_index("s")
    bufs = ((idx_d0, idx_s0, rows0, sem_g0, sem_s0, sem_id0, sem_is0),
            (idx_d1, idx_s1, rows1, sem_g1, sem_s1, sem_id1, sem_is1))
    _zero_rows(rows0, CHS, H)
    r0 = sid * RPT
    nz = RPT // CHS
    for k in range(nz + 1):
        sz = CHS if k < nz else RPT - nz * CHS
        pltpu.sync_copy(rows0.at[pl.ds(0, sz)], acc.at[pl.ds(r0 + k * CHS, sz)])
    plsc.subcore_barrier()

    def start_idx_d(ci, b):
        idx_d, _, _, _, _, sem_id, _ = bufs[b]
        base = sid * EPT + ci * CHS
        pltpu.make_async_copy(dst.at[pl.ds(base, CHS)], idx_d, sem_id).start()

    def wait_idx_d(b):
        idx_d, _, _, _, _, sem_id, _ = bufs[b]
        pltpu.make_async_copy(dst.at[pl.ds(0, CHS)], idx_d, sem_id).wait()

    def start_idx_s(ci, b):
        _, idx_s, _, _, _, _, sem_is = bufs[b]
        base = sid * EPT + ci * CHS
        pltpu.make_async_copy(src.at[pl.ds(base, CHS)], idx_s, sem_is).start()

    def wait_idx_s(b):
        _, idx_s, _, _, _, _, sem_is = bufs[b]
        pltpu.make_async_copy(src.at[pl.ds(0, CHS)], idx_s, sem_is).wait()

    def start_gather(b):
        idx_d, _, rows, sem_g, _, _, _ = bufs[b]

        @pl.when(cid == 0)
        def _():
            pltpu.make_async_copy(tlo.at[idx_d], rows, sem_g).start()

        @pl.when(cid == 1)
        def _():
            pltpu.make_async_copy(thi.at[idx_d], rows, sem_g).start()

    def wait_gather(b):
        idx_d, _, rows, sem_g, _, _, _ = bufs[b]

        @pl.when(cid == 0)
        def _():
            pltpu.make_async_copy(tlo.at[idx_d], rows, sem_g).wait()

        @pl.when(cid == 1)
        def _():
            pltpu.make_async_copy(thi.at[idx_d], rows, sem_g).wait()

    def start_scatter(b):
        _, idx_s, rows, _, sem_s, _, _ = bufs[b]
        pltpu.async_copy(rows, acc.at[idx_s], sem_s, add=True)

    def wait_scatter(b):
        _, idx_s, rows, _, sem_s, _, _ = bufs[b]
        pltpu.make_async_copy(rows, acc.at[idx_s], sem_s).wait()

    NCHK = EPT // CHS
    NPAIR = NCHK // 2
    base0 = sid * EPT
    pltpu.sync_copy(dst.at[pl.ds(base0, CHS)], idx_d0)
    pltpu.sync_copy(src.at[pl.ds(base0, CHS)], idx_s0)
    start_gather(0)
    pltpu.sync_copy(dst.at[pl.ds(base0 + CHS, CHS)], idx_d1)
    pltpu.sync_copy(src.at[pl.ds(base0 + CHS, CHS)], idx_s1)
    start_gather(1)

    def pair(k, _):
        i0 = 2 * k
        wait_gather(0)
        start_idx_d(i0 + 2, 0)

        @pl.when(k > 0)
        def _():
            wait_idx_s(0)

        start_scatter(0)
        wait_gather(1)

        @pl.when(k < NPAIR - 1)
        def _():
            start_idx_d(i0 + 3, 1)

        @pl.when(k > 0)
        def _():
            wait_idx_s(1)

        start_scatter(1)
        wait_scatter(0)
        start_idx_s(i0 + 2, 0)
        wait_idx_d(0)
        start_gather(0)

        @pl.when(k < NPAIR - 1)
        def _():
            wait_scatter(1)
            start_idx_s(i0 + 3, 1)
            wait_idx_d(1)
            start_gather(1)

        return 0

    lax.fori_loop(0, NPAIR, pair, 0)
    wait_gather(0)
    wait_idx_s(0)
    start_scatter(0)
    wait_scatter(1)
    wait_scatter(0)
    plsc.subcore_barrier()

    @pl.when(cid == 0)
    def _():
        pltpu.sync_copy(acc.at[pl.ds(r0, RPT)], outlo.at[pl.ds(r0, RPT)])

    @pl.when(cid == 1)
    def _():
        pltpu.sync_copy(acc.at[pl.ds(r0, RPT)], outhi.at[pl.ds(r0, RPT)])

    @pl.when(sid == NS - 1)
    def _():
        _zero_rows(rows0, CHS, H)
        for m in range(3):
            @pl.when(cid == 0)
            def _():
                pltpu.sync_copy(rows0, outlo.at[pl.ds(N + m * CHS, CHS)])

            @pl.when(cid == 1)
            def _():
                pltpu.sync_copy(rows0, outhi.at[pl.ds(N + m * CHS, CHS)])


_spmm_call = pl.kernel(
    _spmm_body,
    out_type=(
        jax.ShapeDtypeStruct((NP, H), jnp.float32),
        jax.ShapeDtypeStruct((NP, H), jnp.float32),
    ),
    mesh=_sc_mesh,
    compiler_params=_sc_params,
    scratch_types=[
        pltpu.VMEM((CHS,), jnp.int32),
        pltpu.VMEM((CHS,), jnp.int32),
        pltpu.VMEM((CHS, H), jnp.float32),
        pltpu.VMEM((CHS,), jnp.int32),
        pltpu.VMEM((CHS,), jnp.int32),
        pltpu.VMEM((CHS, H), jnp.float32),
        pltpu.SemaphoreType.DMA,
        pltpu.SemaphoreType.DMA,
        pltpu.SemaphoreType.DMA,
        pltpu.SemaphoreType.DMA,
        pltpu.SemaphoreType.DMA,
        pltpu.SemaphoreType.DMA,
        pltpu.SemaphoreType.DMA,
        pltpu.SemaphoreType.DMA,
        pltpu.VMEM_SHARED((N, H), jnp.float32),
    ],
)


def _g16():
    r = lax.broadcasted_iota(jnp.int32, (128, 8), 0) // 16
    c = lax.broadcasted_iota(jnp.int32, (128, 8), 1)
    return (r == c).astype(jnp.float32)


def _prep_kernel(m0_ref, m1_ref, ohk_ref, wpad_ref,
                 c0lo_ref, c0hi_ref, ypool0_ref, ycur0_ref):
    i = pl.program_id(0)
    w = wpad_ref[...]
    u = _norm_rows(jax.nn.relu(w[0:1, :] + w[1:2, :]))
    ul = jnp.concatenate([u[:, :H]] * 4, axis=1)
    uh = jnp.concatenate([u[:, H:]] * 4, axis=1)
    ms = (m0_ref, m1_ref)
    for l in range(LAY):
        me = ms[l][...]
        c0lo_ref[l] = me * ul
        c0hi_ref[l] = me * uh
        cnt = jnp.zeros((B, 1), jnp.float32)
        for k in range(4):
            cnt = cnt + _dot0(ohk_ref[k], me[:, H * k:H * k + 1])

        @pl.when(i == 0)
        def _():
            ypool0_ref[l] = cnt * u

        @pl.when(i != 0)
        def _():
            ypool0_ref[l] += cnt * u

    @pl.when(i == 0)
    def _():
        ycur0_ref[...] = jnp.ones((B, 1), jnp.float32) * u


def _prep_call(m0, m1, ohk, wpad):
    hspec = pl.BlockSpec((BN4, 128), lambda i: (i, 0))
    return pl.pallas_call(
        _prep_kernel,
        grid=(NB,),
        in_specs=[
            hspec, hspec,
            pl.BlockSpec((4, BN4, EMB), lambda i: (0, i, 0)),
            pl.BlockSpec((8, EMB), lambda i: (0, 0)),
        ],
        out_specs=[
            pl.BlockSpec((LAY, BN4, 128), lambda i: (0, i, 0)),
            pl.BlockSpec((LAY, BN4, 128), lambda i: (0, i, 0)),
            pl.BlockSpec((LAY, B, EMB), lambda i: (0, 0, 0)),
            pl.BlockSpec((B, EMB), lambda i: (0, 0)),
        ],
        out_shape=[
            jax.ShapeDtypeStruct((LAY, NP4, 128), jnp.float32),
            jax.ShapeDtypeStruct((LAY, NP4, 128), jnp.float32),
            jax.ShapeDtypeStruct((LAY, B, EMB), jnp.float32),
            jax.ShapeDtypeStruct((B, EMB), jnp.float32),
        ],
    )(m0, m1, ohk, wpad)


def _dense_kernel(nplo_ref, nphi_ref, clo_ref, chi_ref, ohk_ref,
                  w1q_ref, w2q_ref, w1_ref, w2_ref, ypool_ref, ycur_ref,
                  nlo_ref, nhi_ref, ypooln_ref, ycurn_ref):
    i = pl.program_id(0)
    g32 = _g32()
    g32t = _g32t()
    npl = nplo_ref[...]
    nph = nphi_ref[...]
    cl = clo_ref[...]
    chh = chi_ref[...]
    zl = jax.nn.relu(_mm(npl, w1q_ref[0]) + _mm(nph, w1q_ref[2])
                     + _mm(cl, w2q_ref[0]) + _mm(chh, w2q_ref[2]))
    zh = jax.nn.relu(_mm(npl, w1q_ref[1]) + _mm(nph, w1q_ref[3])
                     + _mm(cl, w2q_ref[1]) + _mm(chh, w2q_ref[3]))
    n2 = _mm(zl * zl + zh * zh, g32)
    sc4 = 1.0 / jnp.maximum(jnp.sqrt(n2), _EPS)
    sce = _mm(sc4, g32t)
    nl = zl * sce
    nh = zh * sce
    nlo_ref[...] = nl
    nhi_ref[...] = nh
    yl = jnp.zeros((B, H), jnp.float32)
    yh = jnp.zeros((B, H), jnp.float32)
    for k in range(4):
        yl = yl + _dot0(ohk_ref[k], nl[:, H * k:H * k + H])
        yh = yh + _dot0(ohk_ref[k], nh[:, H * k:H * k + H])
    ypn = jnp.concatenate([yl, yh], axis=1)

    @pl.when(i == 0)
    def _():
        ypooln_ref[...] = ypn
        yz = jax.nn.relu(_mm(ypool_ref[...], w1_ref[...])
                         + _mm(ycur_ref[...], w2_ref[...]))
        ycurn_ref[...] = _norm_rows(yz)

    @pl.when(i != 0)
    def _():
        ypooln_ref[...] += ypn


def _dense_call(nplo, nphi, clo, chi, ohk, w1q, w2q, w1, w2, ypool, ycur):
    bspec = pl.BlockSpec((BN4, 128), lambda i: (i, 0))
    qspec = pl.BlockSpec((4, 128, 128), lambda i: (0, 0, 0))
    wspec = pl.BlockSpec((EMB, EMB), lambda i: (0, 0))
    yspec = pl.BlockSpec((B, EMB), lambda i: (0, 0))
    return pl.pallas_call(
        _dense_kernel,
        grid=(NB,),
        in_specs=[
            bspec, bspec, bspec, bspec,
            pl.BlockSpec((4, BN4, EMB), lambda i: (0, i, 0)),
            qspec, qspec, wspec, wspec, yspec, yspec,
        ],
        out_specs=[bspec, bspec, yspec, yspec],
        out_shape=[
            jax.ShapeDtypeStruct((NP4, 128), jnp.float32),
            jax.ShapeDtypeStruct((NP4, 128), jnp.float32),
            jax.ShapeDtypeStruct((B, EMB), jnp.float32),
            jax.ShapeDtypeStruct((B, EMB), jnp.float32),
        ],
    )(nplo, nphi, clo, chi, ohk, w1q, w2q, w1, w2, ypool, ycur)


def _att_kernel(c0lo_ref, c0hi_ref, c1lo_ref, c1hi_ref, aohk_ref,
                wattq_ref, watt_ref, y0_ref, y1_ref,
                cmlo_ref, cmhi_ref, ymsg_ref, aemb_ref):
    i = pl.program_id(0)
    scale = 1.0 / np.sqrt(EMB)
    g32 = _g32()
    g32t = _g32t()
    e0l = c0lo_ref[...]
    e0h = c0hi_ref[...]
    e1l = c1lo_ref[...]
    e1h = c1hi_ref[...]
    a0l = _mm(e0l, wattq_ref[0]) + _mm(e0h, wattq_ref[2])
    a0h = _mm(e0l, wattq_ref[1]) + _mm(e0h, wattq_ref[3])
    a1l = _mm(e1l, wattq_ref[0]) + _mm(e1h, wattq_ref[2])
    a1h = _mm(e1l, wattq_ref[1]) + _mm(e1h, wattq_ref[3])
    for l, (ell, elh) in ((0, (e0l, e0h)), (1, (e1l, e1h))):
        s0 = _mm(ell * a0l + elh * a0h, g32) * scale
        s1 = _mm(ell * a1l + elh * a1h, g32) * scale
        m = jnp.maximum(s0, s1)
        x0 = jnp.exp(s0 - m)
        x1 = jnp.exp(s1 - m)
        den = x0 + x1
        al0 = _mm(x0 / den, g32t)
        al1 = _mm(x1 / den, g32t)
        ml = al0 * e0l + al1 * e1l
        mh = al0 * e0h + al1 * e1h
        n2 = _mm(ml * ml + mh * mh, g32)
        sc4 = 1.0 / jnp.maximum(jnp.sqrt(n2), _EPS)
        sce = _mm(sc4, g32t)
        cml = ml * sce
        cmh = mh * sce
        cmlo_ref[l] = cml
        cmhi_ref[l] = cmh
        ael = jnp.zeros((B, H), jnp.float32)
        aeh = jnp.zeros((B, H), jnp.float32)
        for k in range(4):
            ael = ael + _dot0(aohk_ref[k], cml[:, H * k:H * k + H])
            aeh = aeh + _dot0(aohk_ref[k], cmh[:, H * k:H * k + H])
        ae = jnp.concatenate([ael, aeh], axis=1)

        @pl.when(i == 0)
        def _():
            aemb_ref[l] = ae

        @pl.when(i != 0)
        def _():
            aemb_ref[l] += ae

    @pl.when(i == 0)
    def _():
        watt = watt_ref[...]
        ye0 = y0_ref[...]
        ye1 = y1_ref[...]
        ya0 = _mm(ye0, watt)
        ya1 = _mm(ye1, watt)
        for l, yel in ((0, ye0), (1, ye1)):
            s0 = jnp.sum(yel * ya0, axis=1, keepdims=True) * scale
            s1 = jnp.sum(yel * ya1, axis=1, keepdims=True) * scale
            m = jnp.maximum(s0, s1)
            x0 = jnp.exp(s0 - m)
            x1 = jnp.exp(s1 - m)
            den = x0 + x1
            ymsg = (x0 / den) * ye0 + (x1 / den) * ye1
            ymsg_ref[l] = _norm_rows(ymsg)


def _att_call(c0lo, c0hi, c1lo, c1hi, aohk, wattq, watt, y0, y1):
    bspec = pl.BlockSpec((BN4, 128), lambda i: (i, 0))
    yspec = pl.BlockSpec((B, EMB), lambda i: (0, 0))
    return pl.pallas_call(
        _att_kernel,
        grid=(NB,),
        in_specs=[
            bspec, bspec, bspec, bspec,
            pl.BlockSpec((4, BN4, EMB), lambda i: (0, i, 0)),
            pl.BlockSpec((4, 128, 128), lambda i: (0, 0, 0)),
            pl.BlockSpec((EMB, EMB), lambda i: (0, 0)),
            yspec, yspec,
        ],
        out_specs=[
            pl.BlockSpec((LAY, BN4, 128), lambda i: (0, i, 0)),
            pl.BlockSpec((LAY, BN4, 128), lambda i: (0, i, 0)),
            pl.BlockSpec((LAY, B, EMB), lambda i: (0, 0, 0)),
            pl.BlockSpec((LAY, B, EMB), lambda i: (0, 0, 0)),
        ],
        out_shape=[
            jax.ShapeDtypeStruct((LAY, NP4, 128), jnp.float32),
            jax.ShapeDtypeStruct((LAY, NP4, 128), jnp.float32),
            jax.ShapeDtypeStruct((LAY, B, EMB), jnp.float32),
            jax.ShapeDtypeStruct((LAY, B, EMB), jnp.float32),
        ],
    )(c0lo, c0hi, c1lo, c1hi, aohk, wattq, watt, y0, y1)


def _head_kernel(aemb_ref, ymsg_ref, aux0_ref, aux1_ref, h1_ref, h2p_ref,
                 crossp_ref, wl1_ref, wl2p_ref, q_ref):
    h1 = h1_ref[...]
    h2 = h2p_ref[...]
    crossp = crossp_ref[...]
    wl1 = wl1_ref[...]
    wl2 = wl2p_ref[...]
    auxs = (aux0_ref[...], aux1_ref[...])
    qs = []
    ws = []
    for l in range(LAY):
        ym = ymsg_ref[l]
        s = _mm(ym, crossp)[:, 0:1]
        esa = aemb_ref[l] * s
        hid = jax.nn.relu(_mm(esa, h1))
        q_l = (_mm(hid, h2[0:RH, :]) + _mm(auxs[l], h2[RH:RH + AUX, :]))[:, 0:1]
        qs.append(q_l)
        wl = _mm(jax.nn.relu(_mm(ym, wl1)), wl2)[:, 0:1]
        ws.append(wl)
    m = jnp.maximum(ws[0], ws[1])
    x0 = jnp.exp(ws[0] - m)
    x1 = jnp.exp(ws[1] - m)
    den = x0 + x1
    q_ref[...] = (x0 / den) * qs[0] + (x1 / den) * qs[1]


def _head_call(aemb, ymsg, aux0, aux1, h1, h2p, crossp, wl1, wl2p):
    return pl.pallas_call(
        _head_kernel,
        out_shape=jax.ShapeDtypeStruct((B, 1), jnp.float32),
    )(aemb, ymsg, aux0, aux1, h1, h2p, crossp, wl1, wl2p)


def _quad(W):
    i4 = jnp.eye(4, dtype=jnp.float32)
    qs = [jnp.kron(i4, W[:H, :H]), jnp.kron(i4, W[:H, H:]),
          jnp.kron(i4, W[H:, :H]), jnp.kron(i4, W[H:, H:])]
    return jnp.stack(qs, axis=0)


def kernel(edge_index, graph_ids, action_nodes, aux_input, w_n2l, p_node_conv,
           p_node_conv2, p_node_conv3, h1_weight, h2_weight, cross_product,
           w_layer1, w_layer2, W_att):
    f32 = jnp.float32
    src0 = edge_index[0, 0]
    dst0 = edge_index[0, 1]
    src1 = edge_index[1, 0]
    dst1 = edge_index[1, 1]

    gid_pad = jnp.concatenate(
        [graph_ids, jnp.full((NP - N,), -1, graph_ids.dtype)])
    gid4 = gid_pad.reshape(NP4, 4).T
    ohk = (gid4[:, :, None] == jnp.arange(B, dtype=gid_pad.dtype)).astype(f32)
    ids4 = jnp.arange(NP, dtype=action_nodes.dtype).reshape(NP4, 4).T
    aohk = (ids4[:, :, None] == action_nodes[None, None, :]).astype(f32)
    w1 = p_node_conv @ p_node_conv3[:EMB]
    w2 = p_node_conv2 @ p_node_conv3[EMB:]
    w1q = _quad(w1)
    w2q = _quad(w2)
    wattq = _quad(W_att)
    wpad = jnp.zeros((8, EMB), f32).at[0:2].set(w_n2l)
    h2p = jnp.zeros((40, 8), f32).at[:RH + AUX, 0:1].set(h2_weight)
    crossp = jnp.zeros((EMB, 8), f32).at[:, 0:1].set(cross_product)
    wl2p = jnp.zeros((128, 8), f32).at[:, 0:1].set(w_layer2)
    aux0 = aux_input[:, 0, :]
    aux1 = aux_input[:, 1, :]

    h00, h01, h10, h11 = _hist_call(src0, src1)
    def _mask32(ha, hb):
        m = ((ha[:, 0] + hb[:, 0]) > 0).astype(f32)
        return jnp.reshape(jnp.tile(m[:, None], (1, H)), (NP4, 128))

    m32_0 = _mask32(h00, h01)
    m32_1 = _mask32(h10, h11)
    c0lo, c0hi, ypool0, ycur0 = _prep_call(m32_0, m32_1, ohk, wpad)

    curs = []
    ycurs = []
    for l, (srcl, dstl) in enumerate(((src0, dst0), (src1, dst1))):
        clo = c0lo[l]
        chi = c0hi[l]
        ypool = ypool0[l]
        ycur = ycur0
        for _ in range(BP):
            nplo_f, nphi_f = _spmm_call(dstl, srcl,
                                        jnp.reshape(clo, (NP, H)),
                                        jnp.reshape(chi, (NP, H)))
            nplo = jnp.reshape(nplo_f, (NP4, 128))
            nphi = jnp.reshape(nphi_f, (NP4, 128))
            clo, chi, ypool, ycur = _dense_call(nplo, nphi, clo, chi, ohk,
                                                w1q, w2q, w1, w2, ypool, ycur)
        curs.append((clo, chi))
        ycurs.append(ycur)

    cmlo, cmhi, ymsg, aemb = _att_call(curs[0][0], curs[0][1],
                                       curs[1][0], curs[1][1],
                                       aohk, wattq, W_att, ycurs[0], ycurs[1])
    q = _head_call(aemb, ymsg, aux0, aux1, h1_weight, h2p, crossp,
                   w_layer1, wl2p)
    cur_msg = jnp.concatenate(
        [cmlo[:, :N // 4, :].reshape(LAY, N, H),
         cmhi[:, :N // 4, :].reshape(LAY, N, H)], axis=2)
    return (q, cur_msg)


# speedup vs baseline: 9.8320x; 1.0005x over previous
import functools

import jax
import jax.numpy as jnp
import numpy as np
from jax import lax
from jax.experimental import pallas as pl
from jax.experimental.pallas import tpu as pltpu
from jax.experimental.pallas import tpu_sc as plsc

N = 50000
B = 64
E = 800000
EMB = 64
AUX = 4
RH = 32
LAY = 2
BP = 3

H = 32
NC = 2
NS = 16
CH = 1000
CHS = 400
RPT = N // NS
EPT = E // NS
EPT2 = E // (NC * NS)

NP = 51200
NP4 = NP // 4
BN4 = 1280
NB = NP4 // BN4

_EPS = 1e-12

_sc_mesh = plsc.VectorSubcoreMesh(core_axis_name="c", subcore_axis_name="s")
_sc_params = pltpu.CompilerParams(use_tc_tiling_on_sc=False,
                                  internal_scratch_in_bytes=0)


def _zero_rows(buf, nrows, ncols):
    zv = jnp.zeros((16,), jnp.float32)

    def body(i, _):
        for j in range(ncols // 16):
            buf[i, pl.ds(j * 16, 16)] = zv
        return 0

    lax.fori_loop(0, nrows, body, 0)


def _fill_ones(buf, nrows, ncols):
    ov = jnp.ones((16,), jnp.float32)

    def body(i, _):
        for j in range(ncols // 16):
            buf[i, pl.ds(j * 16, 16)] = ov
        return 0

    lax.fori_loop(0, nrows, body, 0)


def _g32():
    r = lax.broadcasted_iota(jnp.int32, (128, 4), 0) // H
    c = lax.broadcasted_iota(jnp.int32, (128, 4), 1)
    return (r == c).astype(jnp.float32)


def _g32t():
    r = lax.broadcasted_iota(jnp.int32, (4, 128), 0)
    c = lax.broadcasted_iota(jnp.int32, (4, 128), 1) // H
    return (r == c).astype(jnp.float32)


def _mm(a, b):
    return jnp.dot(a, b, preferred_element_type=jnp.float32)


def _dot0(a, b):
    return lax.dot_general(a, b, (((0,), (0,)), ((), ())),
                           preferred_element_type=jnp.float32)


def _norm_rows(z):
    n = jnp.sqrt(jnp.sum(z * z, axis=1, keepdims=True))
    return z / jnp.maximum(n, _EPS)


def _hist_body(src0, src1, h00, h01, h10, h11, idx_v, ones_v, acc):
    cid = lax.axis_index("c")
    sid = lax.axis_index("s")
    srcs = [src0, src1]
    houts = [[h00, h01], [h10, h11]]
    r0 = sid * RPT
    wid = cid * NS + sid
    for l in range(LAY):
        _zero_rows(ones_v, CH, 16)
        for k in range(4):
            sz = CH if k < 3 else RPT - 3 * CH
            pltpu.sync_copy(ones_v.at[pl.ds(0, sz)], acc.at[pl.ds(r0 + k * CH, sz)])
        _fill_ones(ones_v, CH, 16)
        plsc.subcore_barrier()

        def chunk(ci, _):
            base = wid * EPT2 + ci * CH
            pltpu.sync_copy(srcs[l].at[pl.ds(base, CH)], idx_v)
            pltpu.sync_copy(ones_v, acc.at[idx_v], add=True)
            return 0

        lax.fori_loop(0, EPT2 // CH, chunk, 0)
        plsc.subcore_barrier()

        @pl.when(cid == 0)
        def _():
            pltpu.sync_copy(acc.at[pl.ds(r0, RPT)], houts[l][0].at[pl.ds(r0, RPT)])

        @pl.when(cid == 1)
        def _():
            pltpu.sync_copy(acc.at[pl.ds(r0, RPT)], houts[l][1].at[pl.ds(r0, RPT)])

        plsc.subcore_barrier()


_hist_call = pl.kernel(
    _hist_body,
    out_type=tuple(jax.ShapeDtypeStruct((NP, 16), jnp.float32) for _ in range(4)),
    mesh=_sc_mesh,
    compiler_params=_sc_params,
    scratch_types=[
        pltpu.VMEM((CH,), jnp.int32),
        pltpu.VMEM((CH, 16), jnp.float32),
        pltpu.VMEM_SHARED((N, 16), jnp.float32),
    ],
)


def _spmm_body(dst, src, tlo, thi, outlo, outhi,
               idx_d0, idx_s0, rows0, idx_d1, idx_s1, rows1,
               sem_g0, sem_g1, sem_s0, sem_s1, sem_id0, sem_id1,
               sem_is0, sem_is1, acc):
    cid = lax.axis_index("c")
    sid = lax.axis_index("s")
    bufs = ((idx_d0, idx_s0, rows0, sem_g0, sem_s0, sem_id0, sem_is0),
            (idx_d1, idx_s1, rows1, sem_g1, sem_s1, sem_id1, sem_is1))
    _zero_rows(rows0, CHS, H)
    r0 = sid * RPT
    nz = RPT // CHS
    for k in range(nz + 1):
        sz = CHS if k < nz else RPT - nz * CHS
        pltpu.sync_copy(rows0.at[pl.ds(0, sz)], acc.at[pl.ds(r0 + k * CHS, sz)])
    plsc.subcore_barrier()

    def start_idx_d(ci, b):
        idx_d, _, _, _, _, sem_id, _ = bufs[b]
        base = sid * EPT + ci * CHS
        pltpu.make_async_copy(dst.at[pl.ds(base, CHS)], idx_d, sem_id).start()

    def wait_idx_d(b):
        idx_d, _, _, _, _, sem_id, _ = bufs[b]
        pltpu.make_async_copy(dst.at[pl.ds(0, CHS)], idx_d, sem_id).wait()

    def start_idx_s(ci, b):
        _, idx_s, _, _, _, _, sem_is = bufs[b]
        base = sid * EPT + ci * CHS
        pltpu.make_async_copy(src.at[pl.ds(base, CHS)], idx_s, sem_is).start()

    def wait_idx_s(b):
        _, idx_s, _, _, _, _, sem_is = bufs[b]
        pltpu.make_async_copy(src.at[pl.ds(0, CHS)], idx_s, sem_is).wait()

    def start_gather(b):
        idx_d, _, rows, sem_g, _, _, _ = bufs[b]

        @pl.when(cid == 0)
        def _():
            pltpu.make_async_copy(tlo.at[idx_d], rows, sem_g).start()

        @pl.when(cid == 1)
        def _():
            pltpu.make_async_copy(thi.at[idx_d], rows, sem_g).start()

    def wait_gather(b):
        idx_d, _, rows, sem_g, _, _, _ = bufs[b]

        @pl.when(cid == 0)
        def _():
            pltpu.make_async_copy(tlo.at[idx_d], rows, sem_g).wait()

        @pl.when(cid == 1)
        def _():
            pltpu.make_async_copy(thi.at[idx_d], rows, sem_g).wait()

    def start_scatter(b):
        _, idx_s, rows, _, sem_s, _, _ = bufs[b]
        pltpu.async_copy(rows, acc.at[idx_s], sem_s, add=True)

    def wait_scatter(b):
        _, idx_s, rows, _, sem_s, _, _ = bufs[b]
        pltpu.make_async_copy(rows, acc.at[idx_s], sem_s).wait()

    NCHK = EPT // CHS
    NPAIR = NCHK // 2
    base0 = sid * EPT
    pltpu.sync_copy(dst.at[pl.ds(base0, CHS)], idx_d0)
    pltpu.sync_copy(src.at[pl.ds(base0, CHS)], idx_s0)
    start_gather(0)
    pltpu.sync_copy(dst.at[pl.ds(base0 + CHS, CHS)], idx_d1)
    pltpu.sync_copy(src.at[pl.ds(base0 + CHS, CHS)], idx_s1)
    start_gather(1)

    def pair(k, _):
        i0 = 2 * k
        wait_gather(0)
        start_idx_d(i0 + 2, 0)

        @pl.when(k > 0)
        def _():
            wait_idx_s(0)

        start_scatter(0)
        wait_gather(1)

        @pl.when(k < NPAIR - 1)
        def _():
            start_idx_d(i0 + 3, 1)

        @pl.when(k > 0)
        def _():
            wait_idx_s(1)

        start_scatter(1)
        wait_scatter(0)
        start_idx_s(i0 + 2, 0)
        wait_idx_d(0)
        start_gather(0)

        @pl.when(k < NPAIR - 1)
        def _():
            wait_scatter(1)
            start_idx_s(i0 + 3, 1)
            wait_idx_d(1)
            start_gather(1)

        return 0

    lax.fori_loop(0, NPAIR, pair, 0)
    wait_gather(0)
    wait_idx_s(0)
    start_scatter(0)
    wait_scatter(1)
    wait_scatter(0)
    plsc.subcore_barrier()

    @pl.when(cid == 0)
    def _():
        pltpu.sync_copy(acc.at[pl.ds(r0, RPT)], outlo.at[pl.ds(r0, RPT)])

    @pl.when(cid == 1)
    def _():
        pltpu.sync_copy(acc.at[pl.ds(r0, RPT)], outhi.at[pl.ds(r0, RPT)])

    @pl.when(sid == NS - 1)
    def _():
        _zero_rows(rows0, CHS, H)
        for m in range(3):
            @pl.when(cid == 0)
            def _():
                pltpu.sync_copy(rows0, outlo.at[pl.ds(N + m * CHS, CHS)])

            @pl.when(cid == 1)
            def _():
                pltpu.sync_copy(rows0, outhi.at[pl.ds(N + m * CHS, CHS)])


_spmm_call = pl.kernel(
    _spmm_body,
    out_type=(
        jax.ShapeDtypeStruct((NP, H), jnp.float32),
        jax.ShapeDtypeStruct((NP, H), jnp.float32),
    ),
    mesh=_sc_mesh,
    compiler_params=_sc_params,
    scratch_types=[
        pltpu.VMEM((CHS,), jnp.int32),
        pltpu.VMEM((CHS,), jnp.int32),
        pltpu.VMEM((CHS, H), jnp.float32),
        pltpu.VMEM((CHS,), jnp.int32),
        pltpu.VMEM((CHS,), jnp.int32),
        pltpu.VMEM((CHS, H), jnp.float32),
        pltpu.SemaphoreType.DMA,
        pltpu.SemaphoreType.DMA,
        pltpu.SemaphoreType.DMA,
        pltpu.SemaphoreType.DMA,
        pltpu.SemaphoreType.DMA,
        pltpu.SemaphoreType.DMA,
        pltpu.SemaphoreType.DMA,
        pltpu.SemaphoreType.DMA,
        pltpu.VMEM_SHARED((N, H), jnp.float32),
    ],
)


def _prep_kernel(m0_ref, m1_ref, ohk_ref, wpad_ref,
                 c0lo_ref, c0hi_ref, ypool0_ref, ycur0_ref):
    i = pl.program_id(0)
    w = wpad_ref[...]
    u = _norm_rows(jax.nn.relu(w[0:1, :] + w[1:2, :]))
    ul = jnp.concatenate([u[:, :H]] * 4, axis=1)
    uh = jnp.concatenate([u[:, H:]] * 4, axis=1)
    ms = (m0_ref, m1_ref)
    for l in range(LAY):
        me = ms[l][...]
        c0lo_ref[l] = me * ul
        c0hi_ref[l] = me * uh
        cnt = jnp.zeros((B, 1), jnp.float32)
        for k in range(4):
            cnt = cnt + _dot0(ohk_ref[k], me[:, H * k:H * k + 1])

        @pl.when(i == 0)
        def _():
            ypool0_ref[l] = cnt * u

        @pl.when(i != 0)
        def _():
            ypool0_ref[l] += cnt * u

    @pl.when(i == 0)
    def _():
        ycur0_ref[...] = jnp.ones((B, 1), jnp.float32) * u


def _prep_call(m0, m1, ohk, wpad):
    hspec = pl.BlockSpec((BN4, 128), lambda i: (i, 0))
    return pl.pallas_call(
        _prep_kernel,
        grid=(NB,),
        in_specs=[
            hspec, hspec,
            pl.BlockSpec((4, BN4, EMB), lambda i: (0, i, 0)),
            pl.BlockSpec((8, EMB), lambda i: (0, 0)),
        ],
        out_specs=[
            pl.BlockSpec((LAY, BN4, 128), lambda i: (0, i, 0)),
            pl.BlockSpec((LAY, BN4, 128), lambda i: (0, i, 0)),
            pl.BlockSpec((LAY, B, EMB), lambda i: (0, 0, 0)),
            pl.BlockSpec((B, EMB), lambda i: (0, 0)),
        ],
        out_shape=[
            jax.ShapeDtypeStruct((LAY, NP4, 128), jnp.float32),
            jax.ShapeDtypeStruct((LAY, NP4, 128), jnp.float32),
            jax.ShapeDtypeStruct((LAY, B, EMB), jnp.float32),
            jax.ShapeDtypeStruct((B, EMB), jnp.float32),
        ],
    )(m0, m1, ohk, wpad)


def _dense_kernel(nplo_ref, nphi_ref, clo_ref, chi_ref, ohk_ref,
                  w1q_ref, w2q_ref, w1_ref, w2_ref, ypool_ref, ycur_ref,
                  nlo_ref, nhi_ref, ypooln_ref, ycurn_ref):
    i = pl.program_id(0)
    g32 = _g32()
    g32t = _g32t()
    npl = nplo_ref[...]
    nph = nphi_ref[...]
    cl = clo_ref[...]
    chh = chi_ref[...]
    zl = jax.nn.relu(_mm(npl, w1q_ref[0]) + _mm(nph, w1q_ref[2])
                     + _mm(cl, w2q_ref[0]) + _mm(chh, w2q_ref[2]))
    zh = jax.nn.relu(_mm(npl, w1q_ref[1]) + _mm(nph, w1q_ref[3])
                     + _mm(cl, w2q_ref[1]) + _mm(chh, w2q_ref[3]))
    n2 = _mm(zl * zl + zh * zh, g32)
    sc4 = 1.0 / jnp.maximum(jnp.sqrt(n2), _EPS)
    sce = _mm(sc4, g32t)
    nl = zl * sce
    nh = zh * sce
    nlo_ref[...] = nl
    nhi_ref[...] = nh
    yl = jnp.zeros((B, H), jnp.float32)
    yh = jnp.zeros((B, H), jnp.float32)
    for k in range(4):
        yl = yl + _dot0(ohk_ref[k], nl[:, H * k:H * k + H])
        yh = yh + _dot0(ohk_ref[k], nh[:, H * k:H * k + H])
    ypn = jnp.concatenate([yl, yh], axis=1)

    @pl.when(i == 0)
    def _():
        ypooln_ref[...] = ypn
        yz = jax.nn.relu(_mm(ypool_ref[...], w1_ref[...])
                         + _mm(ycur_ref[...], w2_ref[...]))
        ycurn_ref[...] = _norm_rows(yz)

    @pl.when(i != 0)
    def _():
        ypooln_ref[...] += ypn


def _dense_call(nplo, nphi, clo, chi, ohk, w1q, w2q, w1, w2, ypool, ycur):
    bspec = pl.BlockSpec((BN4, 128), lambda i: (i, 0))
    qspec = pl.BlockSpec((4, 128, 128), lambda i: (0, 0, 0))
    wspec = pl.BlockSpec((EMB, EMB), lambda i: (0, 0))
    yspec = pl.BlockSpec((B, EMB), lambda i: (0, 0))
    return pl.pallas_call(
        _dense_kernel,
        grid=(NB,),
        in_specs=[
            bspec, bspec, bspec, bspec,
            pl.BlockSpec((4, BN4, EMB), lambda i: (0, i, 0)),
            qspec, qspec, wspec, wspec, yspec, yspec,
        ],
        out_specs=[bspec, bspec, yspec, yspec],
        out_shape=[
            jax.ShapeDtypeStruct((NP4, 128), jnp.float32),
            jax.ShapeDtypeStruct((NP4, 128), jnp.float32),
            jax.ShapeDtypeStruct((B, EMB), jnp.float32),
            jax.ShapeDtypeStruct((B, EMB), jnp.float32),
        ],
    )(nplo, nphi, clo, chi, ohk, w1q, w2q, w1, w2, ypool, ycur)


def _att_kernel(c0lo_ref, c0hi_ref, c1lo_ref, c1hi_ref, aohk_ref,
                wattq_ref, watt_ref, y0_ref, y1_ref,
                cmlo_ref, cmhi_ref, ymsg_ref, aemb_ref):
    i = pl.program_id(0)
    scale = 1.0 / np.sqrt(EMB)
    g32 = _g32()
    g32t = _g32t()
    e0l = c0lo_ref[...]
    e0h = c0hi_ref[...]
    e1l = c1lo_ref[...]
    e1h = c1hi_ref[...]
    a0l = _mm(e0l, wattq_ref[0]) + _mm(e0h, wattq_ref[2])
    a0h = _mm(e0l, wattq_ref[1]) + _mm(e0h, wattq_ref[3])
    a1l = _mm(e1l, wattq_ref[0]) + _mm(e1h, wattq_ref[2])
    a1h = _mm(e1l, wattq_ref[1]) + _mm(e1h, wattq_ref[3])
    for l, (ell, elh) in ((0, (e0l, e0h)), (1, (e1l, e1h))):
        s0 = _mm(ell * a0l + elh * a0h, g32) * scale
        s1 = _mm(ell * a1l + elh * a1h, g32) * scale
        m = jnp.maximum(s0, s1)
        x0 = jnp.exp(s0 - m)
        x1 = jnp.exp(s1 - m)
        den = x0 + x1
        al0 = _mm(x0 / den, g32t)
        al1 = _mm(x1 / den, g32t)
        ml = al0 * e0l + al1 * e1l
        mh = al0 * e0h + al1 * e1h
        n2 = _mm(ml * ml + mh * mh, g32)
        sc4 = 1.0 / jnp.maximum(jnp.sqrt(n2), _EPS)
        sce = _mm(sc4, g32t)
        cml = ml * sce
        cmh = mh * sce
        cmlo_ref[l] = cml
        cmhi_ref[l] = cmh
        ael = jnp.zeros((B, H), jnp.float32)
        aeh = jnp.zeros((B, H), jnp.float32)
        for k in range(4):
            ael = ael + _dot0(aohk_ref[k], cml[:, H * k:H * k + H])
            aeh = aeh + _dot0(aohk_ref[k], cmh[:, H * k:H * k + H])
        ae = jnp.concatenate([ael, aeh], axis=1)

        @pl.when(i == 0)
        def _():
            aemb_ref[l] = ae

        @pl.when(i != 0)
        def _():
            aemb_ref[l] += ae

    @pl.when(i == 0)
    def _():
        watt = watt_ref[...]
        ye0 = y0_ref[...]
        ye1 = y1_ref[...]
        ya0 = _mm(ye0, watt)
        ya1 = _mm(ye1, watt)
        for l, yel in ((0, ye0), (1, ye1)):
            s0 = jnp.sum(yel * ya0, axis=1, keepdims=True) * scale
            s1 = jnp.sum(yel * ya1, axis=1, keepdims=True) * scale
            m = jnp.maximum(s0, s1)
            x0 = jnp.exp(s0 - m)
            x1 = jnp.exp(s1 - m)
            den = x0 + x1
            ymsg = (x0 / den) * ye0 + (x1 / den) * ye1
            ymsg_ref[l] = _norm_rows(ymsg)


def _att_call(c0lo, c0hi, c1lo, c1hi, aohk, wattq, watt, y0, y1):
    bspec = pl.BlockSpec((BN4, 128), lambda i: (i, 0))
    yspec = pl.BlockSpec((B, EMB), lambda i: (0, 0))
    return pl.pallas_call(
        _att_kernel,
        grid=(NB,),
        in_specs=[
            bspec, bspec, bspec, bspec,
            pl.BlockSpec((4, BN4, EMB), lambda i: (0, i, 0)),
            pl.BlockSpec((4, 128, 128), lambda i: (0, 0, 0)),
            pl.BlockSpec((EMB, EMB), lambda i: (0, 0)),
            yspec, yspec,
        ],
        out_specs=[
            pl.BlockSpec((LAY, BN4, 128), lambda i: (0, i, 0)),
            pl.BlockSpec((LAY, BN4, 128), lambda i: (0, i, 0)),
            pl.BlockSpec((LAY, B, EMB), lambda i: (0, 0, 0)),
            pl.BlockSpec((LAY, B, EMB), lambda i: (0, 0, 0)),
        ],
        out_shape=[
            jax.ShapeDtypeStruct((LAY, NP4, 128), jnp.float32),
            jax.ShapeDtypeStruct((LAY, NP4, 128), jnp.float32),
            jax.ShapeDtypeStruct((LAY, B, EMB), jnp.float32),
            jax.ShapeDtypeStruct((LAY, B, EMB), jnp.float32),
        ],
    )(c0lo, c0hi, c1lo, c1hi, aohk, wattq, watt, y0, y1)


def _head_kernel(aemb_ref, ymsg_ref, aux0_ref, aux1_ref, h1_ref, h2p_ref,
                 crossp_ref, wl1_ref, wl2p_ref, q_ref):
    h1 = h1_ref[...]
    h2 = h2p_ref[...]
    crossp = crossp_ref[...]
    wl1 = wl1_ref[...]
    wl2 = wl2p_ref[...]
    auxs = (aux0_ref[...], aux1_ref[...])
    qs = []
    ws = []
    for l in range(LAY):
        ym = ymsg_ref[l]
        s = _mm(ym, crossp)[:, 0:1]
        esa = aemb_ref[l] * s
        hid = jax.nn.relu(_mm(esa, h1))
        q_l = (_mm(hid, h2[0:RH, :]) + _mm(auxs[l], h2[RH:RH + AUX, :]))[:, 0:1]
        qs.append(q_l)
        wl = _mm(jax.nn.relu(_mm(ym, wl1)), wl2)[:, 0:1]
        ws.append(wl)
    m = jnp.maximum(ws[0], ws[1])
    x0 = jnp.exp(ws[0] - m)
    x1 = jnp.exp(ws[1] - m)
    den = x0 + x1
    q_ref[...] = (x0 / den) * qs[0] + (x1 / den) * qs[1]


def _head_call(aemb, ymsg, aux0, aux1, h1, h2p, crossp, wl1, wl2p):
    return pl.pallas_call(
        _head_kernel,
        out_shape=jax.ShapeDtypeStruct((B, 1), jnp.float32),
    )(aemb, ymsg, aux0, aux1, h1, h2p, crossp, wl1, wl2p)


def _quad(W):
    i4 = jnp.eye(4, dtype=jnp.float32)
    qs = [jnp.kron(i4, W[:H, :H]), jnp.kron(i4, W[:H, H:]),
          jnp.kron(i4, W[H:, :H]), jnp.kron(i4, W[H:, H:])]
    return jnp.stack(qs, axis=0)


def kernel(edge_index, graph_ids, action_nodes, aux_input, w_n2l, p_node_conv,
           p_node_conv2, p_node_conv3, h1_weight, h2_weight, cross_product,
           w_layer1, w_layer2, W_att):
    f32 = jnp.float32
    src0 = edge_index[0, 0]
    dst0 = edge_index[0, 1]
    src1 = edge_index[1, 0]
    dst1 = edge_index[1, 1]

    gid_pad = jnp.concatenate(
        [graph_ids, jnp.full((NP - N,), -1, graph_ids.dtype)])
    gid4 = gid_pad.reshape(NP4, 4).T
    ohk = (gid4[:, :, None] == jnp.arange(B, dtype=gid_pad.dtype)).astype(f32)
    ids4 = jnp.arange(NP, dtype=action_nodes.dtype).reshape(NP4, 4).T
    aohk = (ids4[:, :, None] == action_nodes[None, None, :]).astype(f32)
    w1 = p_node_conv @ p_node_conv3[:EMB]
    w2 = p_node_conv2 @ p_node_conv3[EMB:]
    w1q = _quad(w1)
    w2q = _quad(w2)
    wattq = _quad(W_att)
    wpad = jnp.zeros((8, EMB), f32).at[0:2].set(w_n2l)
    h2p = jnp.zeros((40, 8), f32).at[:RH + AUX, 0:1].set(h2_weight)
    crossp = jnp.zeros((EMB, 8), f32).at[:, 0:1].set(cross_product)
    wl2p = jnp.zeros((128, 8), f32).at[:, 0:1].set(w_layer2)
    aux0 = aux_input[:, 0, :]
    aux1 = aux_input[:, 1, :]

    h00, h01, h10, h11 = _hist_call(src0, src1)
    def _mask32(ha, hb):
        m = ((ha[:, 0] + hb[:, 0]) > 0).astype(f32)
        return jnp.reshape(jnp.tile(m[:, None], (1, H)), (NP4, 128))

    m32_0 = _mask32(h00, h01)
    m32_1 = _mask32(h10, h11)
    c0lo, c0hi, ypool0, ycur0 = _prep_call(m32_0, m32_1, ohk, wpad)

    curs = []
    ycurs = []
    for l, (srcl, dstl) in enumerate(((src0, dst0), (src1, dst1))):
        clo = c0lo[l]
        chi = c0hi[l]
        ypool = ypool0[l]
        ycur = ycur0
        for _ in range(BP):
            nplo_f, nphi_f = _spmm_call(dstl, srcl,
                                        jnp.reshape(clo, (NP, H)),
                                        jnp.reshape(chi, (NP, H)))
            nplo = jnp.reshape(nplo_f, (NP4, 128))
            nphi = jnp.reshape(nphi_f, (NP4, 128))
            clo, chi, ypool, ycur = _dense_call(nplo, nphi, clo, chi, ohk,
                                                w1q, w2q, w1, w2, ypool, ycur)
        curs.append((clo, chi))
        ycurs.append(ycur)

    cmlo, cmhi, ymsg, aemb = _att_call(curs[0][0], curs[0][1],
                                       curs[1][0], curs[1][1],
                                       aohk, wattq, W_att, ycurs[0], ycurs[1])
    q = _head_call(aemb, ymsg, aux0, aux1, h1_weight, h2p, crossp,
                   w_layer1, wl2p)
    cur_msg = jnp.concatenate(
        [cmlo[:, :N // 4, :].reshape(LAY, N, H),
         cmhi[:, :N // 4, :].reshape(LAY, N, H)], axis=2)
    return (q, cur_msg)
